# Initial kernel scaffold; baseline (speedup 1.0000x reference)
import jax
import jax.numpy as jnp
from jax.experimental import pallas as pl


def kernel(x, edge_index, W_emb, b_emb, Wq, bq, Wk, bk, Wv, bv, Ws, bs):
    raise NotImplementedError("write your pallas kernel here")


# speedup vs baseline: 2.1174x; 2.1174x over previous
import jax
import jax.numpy as jnp
import numpy as np
from jax import lax
from jax.experimental import pallas as pl
from jax.experimental.pallas import tpu as pltpu
from jax.experimental.pallas import tpu_sc as plsc

N = 10000
E = 320000
D_IN = 128
H = 256
N_STEPS = 4
NP = 10240
NC, NS = 2, 16
NT = NC * NS
SCALE = 0.0625


def _matmul_bias(xa, w, b, bm=2000):
    n, kd = xa.shape
    m = w.shape[1]

    def body(x_ref, w_ref, b_ref, o_ref):
        o_ref[...] = (
            jnp.dot(x_ref[...], w_ref[...], preferred_element_type=jnp.float32)
            + b_ref[...]
        )

    return pl.pallas_call(
        body,
        grid=(n // bm,),
        in_specs=[
            pl.BlockSpec((bm, kd), lambda i: (i, 0)),
            pl.BlockSpec((kd, m), lambda i: (0, 0)),
            pl.BlockSpec((1, m), lambda i: (0, 0)),
        ],
        out_specs=pl.BlockSpec((bm, m), lambda i: (i, 0)),
        out_shape=jax.ShapeDtypeStruct((n, m), jnp.float32),
    )(xa, w, b.reshape(1, m))


def _qkvs(y, wcat, bcat, bm=2000):
    def body(y_ref, w_ref, b_ref, q_ref, k_ref, v0_ref, v1_ref, s_ref):
        acc = (
            jnp.dot(y_ref[...], w_ref[...], preferred_element_type=jnp.float32)
            + b_ref[...]
        )
        q_ref[...] = acc[:, 0:256]
        k_ref[...] = acc[:, 256:512]
        v0_ref[...] = acc[:, 512:640]
        v1_ref[...] = acc[:, 640:768]
        s_ref[...] = acc[:, 768:1024]

    grid = (N // bm,)
    return pl.pallas_call(
        body,
        grid=grid,
        in_specs=[
            pl.BlockSpec((bm, H), lambda i: (i, 0)),
            pl.BlockSpec((H, 4 * H), lambda i: (0, 0)),
            pl.BlockSpec((1, 4 * H), lambda i: (0, 0)),
        ],
        out_specs=[
            pl.BlockSpec((bm, H), lambda i: (i, 0)),
            pl.BlockSpec((bm, H), lambda i: (i, 0)),
            pl.BlockSpec((bm, H // 2), lambda i: (i, 0)),
            pl.BlockSpec((bm, H // 2), lambda i: (i, 0)),
            pl.BlockSpec((bm, H), lambda i: (i, 0)),
        ],
        out_shape=[
            jax.ShapeDtypeStruct((N, H), jnp.float32),
            jax.ShapeDtypeStruct((N, H), jnp.float32),
            jax.ShapeDtypeStruct((N, H // 2), jnp.float32),
            jax.ShapeDtypeStruct((N, H // 2), jnp.float32),
            jax.ShapeDtypeStruct((N, H), jnp.float32),
        ],
    )(y, wcat, bcat.reshape(1, 4 * H))


def _euler_update(y, aggn, s, dt, bm=2000):
    def body(y_ref, a_ref, s_ref, o_ref):
        o_ref[...] = y_ref[...] + dt * (a_ref[...] + s_ref[...])

    return pl.pallas_call(
        body,
        grid=(N // bm,),
        in_specs=[pl.BlockSpec((bm, H), lambda i: (i, 0))] * 3,
        out_specs=pl.BlockSpec((bm, H), lambda i: (i, 0)),
        out_shape=jax.ShapeDtypeStruct((N, H), jnp.float32),
    )(y, aggn, s)


_CHUNK_A = 80
_EPT_A = E // NT
_CHUNK_B = 80
_EPT_B = E // NS
_RPT = NP // NS
_RCH = 64
_HH = H // 2


def _score_body(q_hbm, k_hbm, src_hbm, dst_hbm, scores_hbm, pmax_hbm,
                dsti, srci, qrows, krows, sc_v, mx_v, sem):
    wid = lax.axis_index("s") * NC + lax.axis_index("c")
    base = wid * _EPT_A
    lanes = lax.iota(jnp.int32, 16)

    def chunk(ci, mv):
        off = base + ci * _CHUNK_A
        pltpu.sync_copy(dst_hbm.at[pl.ds(off, _CHUNK_A)], dsti)
        pltpu.sync_copy(src_hbm.at[pl.ds(off, _CHUNK_A)], srci)
        cq = pltpu.async_copy(q_hbm.at[dsti], qrows, sem)
        ck = pltpu.async_copy(k_hbm.at[srci], krows, sem)
        cq.wait()
        ck.wait()

        def grp(g, mcur):
            eidx = lanes + g * 16
            acc = jnp.zeros((16,), jnp.float32)
            for j in range(H):
                cidx = jnp.full((16,), j, jnp.int32)
                acc = acc + (plsc.load_gather(qrows, [eidx, cidx])
                             * plsc.load_gather(krows, [eidx, cidx]))
            sc = acc * SCALE
            sc_v[pl.ds(g * 16, 16)] = sc
            return jnp.maximum(mcur, sc)

        mv = lax.fori_loop(0, _CHUNK_A // 16, grp, mv)
        pltpu.sync_copy(sc_v, scores_hbm.at[pl.ds(off, _CHUNK_A)])
        return mv

    mv = lax.fori_loop(0, _EPT_A // _CHUNK_A, chunk,
                       jnp.full((16,), -3.0e38, jnp.float32))
    mx_v[...] = mv
    pltpu.sync_copy(mx_v, pmax_hbm.at[pl.ds(wid * 16, 16)])


def _sc_scores(q, k, src, dst):
    mesh = plsc.VectorSubcoreMesh(core_axis_name="c", subcore_axis_name="s",
                                  num_cores=NC, num_subcores=NS)
    kern = pl.kernel(
        _score_body,
        out_type=[
            jax.ShapeDtypeStruct((E,), jnp.float32),
            jax.ShapeDtypeStruct((NT * 16,), jnp.float32),
        ],
        mesh=mesh,
        compiler_params=pltpu.CompilerParams(use_tc_tiling_on_sc=False, needs_layout_passes=False),
        scratch_types=[
            pltpu.VMEM((_CHUNK_A,), jnp.int32),
            pltpu.VMEM((_CHUNK_A,), jnp.int32),
            pltpu.VMEM((_CHUNK_A, H), jnp.float32),
            pltpu.VMEM((_CHUNK_A, H), jnp.float32),
            pltpu.VMEM((_CHUNK_A,), jnp.float32),
            pltpu.VMEM((16,), jnp.float32),
            pltpu.SemaphoreType.DMA,
        ],
    )
    return kern(q, k, src, dst)


def _agg_body(scores_hbm, pmax_hbm, src_hbm, dst_hbm, vflat_hbm, out_hbm,
              srci, dsti, idxc, sc_v, ev_v, vrows, arows, den_v, pm_v,
              agg_sh, den_sh, sem):
    c = lax.axis_index("c")
    t = lax.axis_index("s")

    def zrow(i, _):
        for j in range(_HH // 16):
            arows[i, pl.ds(j * 16, 16)] = jnp.zeros((16,), jnp.float32)
        return 0

    lax.fori_loop(0, _RCH, zrow, 0)

    def zev(g, _):
        ev_v[pl.ds(g * 16, 16)] = jnp.zeros((16,), jnp.float32)
        return 0

    lax.fori_loop(0, _CHUNK_B // 16, zev, 0)

    def zagg(rc, _):
        pltpu.sync_copy(arows, agg_sh.at[pl.ds(t * _RPT + rc * _RCH, _RCH)])
        return 0

    lax.fori_loop(0, _RPT // _RCH, zagg, 0)

    def zden(zi, _):
        pltpu.sync_copy(ev_v, den_sh.at[pl.ds(t * _RPT + zi * _CHUNK_B, _CHUNK_B)])
        return 0

    lax.fori_loop(0, _RPT // _CHUNK_B, zden, 0)
    plsc.subcore_barrier()

    pltpu.sync_copy(pmax_hbm, pm_v)
    mv = pm_v[pl.ds(0, 16)]
    for g in range(1, NT):
        mv = jnp.maximum(mv, pm_v[pl.ds(g * 16, 16)])
    gmax = jnp.max(mv)

    ebase = t * _EPT_B
    coff = jnp.full((16,), c * N, jnp.int32)

    def chunk(ci, _):
        off = ebase + ci * _CHUNK_B
        pltpu.sync_copy(src_hbm.at[pl.ds(off, _CHUNK_B)], srci)
        pltpu.sync_copy(dst_hbm.at[pl.ds(off, _CHUNK_B)], dsti)
        pltpu.sync_copy(scores_hbm.at[pl.ds(off, _CHUNK_B)], sc_v)

        def vec(g, _):
            sl = pl.ds(g * 16, 16)
            ev_v[sl] = jnp.exp(sc_v[sl] - gmax)
            idxc[sl] = srci[sl] + coff
            return 0

        lax.fori_loop(0, _CHUNK_B // 16, vec, 0)
        pltpu.async_copy(vflat_hbm.at[idxc], vrows, sem).wait()

        def edge(e, _):
            eb = plsc.load_gather(ev_v, [jnp.full((16,), e, jnp.int32)])
            for j in range(_HH // 16):
                sl = pl.ds(j * 16, 16)
                vrows[e, sl] = vrows[e, sl] * eb
            return 0

        lax.fori_loop(0, _CHUNK_B, edge, 0)
        pltpu.sync_copy(vrows, agg_sh.at[dsti], add=True)
        pltpu.sync_copy(ev_v, den_sh.at[dsti], add=True)
        return 0

    lax.fori_loop(0, _EPT_B // _CHUNK_B, chunk, 0)
    plsc.subcore_barrier()

    rbase = t * _RPT

    def nchunk(rc, _):
        r0 = rbase + rc * _RCH
        pltpu.sync_copy(agg_sh.at[pl.ds(r0, _RCH)], arows)
        pltpu.sync_copy(den_sh.at[pl.ds(r0, _RCH)], den_v)

        def row(r, _):
            db = plsc.load_gather(den_v, [jnp.full((16,), r, jnp.int32)]) + 1e-16
            for j in range(_HH // 16):
                sl = pl.ds(j * 16, 16)
                arows[r, sl] = arows[r, sl] / db
            return 0

        lax.fori_loop(0, _RCH, row, 0)
        pltpu.sync_copy(arows, out_hbm.at[pl.ds(c * NP + r0, _RCH)])
        return 0

    lax.fori_loop(0, _RPT // _RCH, nchunk, 0)


def _sc_aggregate(scores, pmax, src, dst, vflat):
    mesh = plsc.VectorSubcoreMesh(core_axis_name="c", subcore_axis_name="s",
                                  num_cores=NC, num_subcores=NS)
    kern = pl.kernel(
        _agg_body,
        out_type=jax.ShapeDtypeStruct((2 * NP, _HH), jnp.float32),
        mesh=mesh,
        compiler_params=pltpu.CompilerParams(use_tc_tiling_on_sc=False, needs_layout_passes=False),
        scratch_types=[
            pltpu.VMEM((_CHUNK_B,), jnp.int32),
            pltpu.VMEM((_CHUNK_B,), jnp.int32),
            pltpu.VMEM((_CHUNK_B,), jnp.int32),
            pltpu.VMEM((_CHUNK_B,), jnp.float32),
            pltpu.VMEM((_CHUNK_B,), jnp.float32),
            pltpu.VMEM((_CHUNK_B, _HH), jnp.float32),
            pltpu.VMEM((_RCH, _HH), jnp.float32),
            pltpu.VMEM((_RCH,), jnp.float32),
            pltpu.VMEM((NT * 16,), jnp.float32),
            pltpu.VMEM_SHARED((NP, _HH), jnp.float32),
            pltpu.VMEM_SHARED((NP,), jnp.float32),
            pltpu.SemaphoreType.DMA,
        ],
    )
    return kern(scores, pmax, src, dst, vflat)


def kernel(x, edge_index, W_emb, b_emb, Wq, bq, Wk, bk, Wv, bv, Ws, bs):
    src = edge_index[0]
    dst = edge_index[1]
    h = _matmul_bias(x, W_emb, b_emb)

    wcat = jnp.concatenate([Wq[1:], Wk[1:], Wv[1:], Ws[1:]], axis=1)
    ts = np.linspace(0.0, 1.0, N_STEPS).astype(np.float32)

    ys = [h]
    y = h
    for i in range(N_STEPS - 1):
        tcur = float(ts[i])
        dt = float(ts[i + 1] - ts[i])
        bcat = jnp.concatenate(
            [bq + tcur * Wq[0], bk + tcur * Wk[0],
             bv + tcur * Wv[0], bs + tcur * Ws[0]]
        )
        q, k, v0, v1, s = _qkvs(y, wcat, bcat)
        vflat = jnp.concatenate([v0, v1], axis=0)
        scores, pmax = _sc_scores(q, k, src, dst)
        aggflat = _sc_aggregate(scores, pmax, src, dst, vflat)
        aggn = jnp.concatenate([aggflat[:N], aggflat[NP:NP + N]], axis=1)
        y = _euler_update(y, aggn, s, dt)
        ys.append(y)
    return jnp.stack(ys, axis=0)


# speedup vs baseline: 2.4580x; 1.1608x over previous
import jax
import jax.numpy as jnp
import numpy as np
from jax import lax
from jax.experimental import pallas as pl
from jax.experimental.pallas import tpu as pltpu
from jax.experimental.pallas import tpu_sc as plsc

N = 10000
E = 320000
D_IN = 128
H = 256
N_STEPS = 4
NP = 10240
NC, NS = 2, 16
NT = NC * NS
SCALE = 0.0625


def _matmul_bias(xa, w, b, bm=2000):
    n, kd = xa.shape
    m = w.shape[1]

    def body(x_ref, w_ref, b_ref, o_ref):
        o_ref[...] = (
            jnp.dot(x_ref[...], w_ref[...], preferred_element_type=jnp.float32)
            + b_ref[...]
        )

    return pl.pallas_call(
        body,
        grid=(n // bm,),
        in_specs=[
            pl.BlockSpec((bm, kd), lambda i: (i, 0)),
            pl.BlockSpec((kd, m), lambda i: (0, 0)),
            pl.BlockSpec((1, m), lambda i: (0, 0)),
        ],
        out_specs=pl.BlockSpec((bm, m), lambda i: (i, 0)),
        out_shape=jax.ShapeDtypeStruct((n, m), jnp.float32),
    )(xa, w, b.reshape(1, m))


def _qkvs(y, wcat, bcat, bm=2000):
    def body(y_ref, w_ref, b_ref, q_ref, k_ref, v0_ref, v1_ref, s_ref):
        acc = (
            jnp.dot(y_ref[...], w_ref[...], preferred_element_type=jnp.float32)
            + b_ref[...]
        )
        q_ref[...] = acc[:, 0:256]
        k_ref[...] = acc[:, 256:512]
        v0_ref[...] = acc[:, 512:640]
        v1_ref[...] = acc[:, 640:768]
        s_ref[...] = acc[:, 768:1024]

    grid = (N // bm,)
    return pl.pallas_call(
        body,
        grid=grid,
        in_specs=[
            pl.BlockSpec((bm, H), lambda i: (i, 0)),
            pl.BlockSpec((H, 4 * H), lambda i: (0, 0)),
            pl.BlockSpec((1, 4 * H), lambda i: (0, 0)),
        ],
        out_specs=[
            pl.BlockSpec((bm, H), lambda i: (i, 0)),
            pl.BlockSpec((bm, H), lambda i: (i, 0)),
            pl.BlockSpec((bm, H // 2), lambda i: (i, 0)),
            pl.BlockSpec((bm, H // 2), lambda i: (i, 0)),
            pl.BlockSpec((bm, H), lambda i: (i, 0)),
        ],
        out_shape=[
            jax.ShapeDtypeStruct((N, H), jnp.float32),
            jax.ShapeDtypeStruct((N, H), jnp.float32),
            jax.ShapeDtypeStruct((N, H // 2), jnp.float32),
            jax.ShapeDtypeStruct((N, H // 2), jnp.float32),
            jax.ShapeDtypeStruct((N, H), jnp.float32),
        ],
    )(y, wcat, bcat.reshape(1, 4 * H))


def _euler_update(y, aggn, s, dt, bm=2000):
    def body(y_ref, a_ref, s_ref, o_ref):
        o_ref[...] = y_ref[...] + dt * (a_ref[...] + s_ref[...])

    return pl.pallas_call(
        body,
        grid=(N // bm,),
        in_specs=[pl.BlockSpec((bm, H), lambda i: (i, 0))] * 3,
        out_specs=pl.BlockSpec((bm, H), lambda i: (i, 0)),
        out_shape=jax.ShapeDtypeStruct((N, H), jnp.float32),
    )(y, aggn, s)


_CHUNK_A = 80
_EPT_A = E // NT
_CHUNK_B = 80
_EPT_B = E // NS
_RPT = NP // NS
_RCH = 64
_HH = H // 2


def _score_body(q_hbm, k_hbm, ei_hbm, scores_hbm, pmax_hbm,
                ei0, ei1, qr0, qr1, kr0, kr1, sv0, sv1, mx_v,
                sem0, sem1, semw):
    wid = lax.axis_index("s") * NC + lax.axis_index("c")
    base = wid * _EPT_A
    lanes = lax.iota(jnp.int32, 16)
    ones = jnp.full((16,), 1, jnp.int32)
    eib = (ei0, ei1)
    qrb = (qr0, qr1)
    krb = (kr0, kr1)
    svb = (sv0, sv1)
    semb = (sem0, sem1)
    nch = _EPT_A // _CHUNK_A

    def fire(cur, b):
        off = base + cur * _CHUNK_A
        pltpu.sync_copy(ei_hbm.at[:, pl.ds(off, _CHUNK_A)], eib[b])
        pltpu.async_copy(q_hbm.at[eib[b].at[1]], qrb[b], semb[b])
        pltpu.async_copy(k_hbm.at[eib[b].at[0]], krb[b], semb[b])

    def wait_gathers(b):
        pltpu.make_async_copy(q_hbm.at[pl.ds(0, _CHUNK_A)], qrb[b], semb[b]).wait()
        pltpu.make_async_copy(k_hbm.at[pl.ds(0, _CHUNK_A)], krb[b], semb[b]).wait()

    def compute(cur, b, mv):
        off = base + cur * _CHUNK_A

        @pl.when(cur >= 2)
        def _():
            pltpu.make_async_copy(svb[b], scores_hbm.at[pl.ds(0, _CHUNK_A)],
                                  semw).wait()

        def grp(g, mcur):
            eidx = lanes + g * 16
            cidx = jnp.zeros((16,), jnp.int32)
            a0 = jnp.zeros((16,), jnp.float32)
            a1 = jnp.zeros((16,), jnp.float32)
            a2 = jnp.zeros((16,), jnp.float32)
            a3 = jnp.zeros((16,), jnp.float32)
            for j in range(H // 4):
                c0 = cidx
                c1 = c0 + ones
                c2 = c1 + ones
                c3 = c2 + ones
                a0 = a0 + (plsc.load_gather(qrb[b], [eidx, c0])
                           * plsc.load_gather(krb[b], [eidx, c0]))
                a1 = a1 + (plsc.load_gather(qrb[b], [eidx, c1])
                           * plsc.load_gather(krb[b], [eidx, c1]))
                a2 = a2 + (plsc.load_gather(qrb[b], [eidx, c2])
                           * plsc.load_gather(krb[b], [eidx, c2]))
                a3 = a3 + (plsc.load_gather(qrb[b], [eidx, c3])
                           * plsc.load_gather(krb[b], [eidx, c3]))
                cidx = c3 + ones
            sc = ((a0 + a1) + (a2 + a3)) * SCALE
            svb[b][pl.ds(g * 16, 16)] = sc
            return jnp.maximum(mcur, sc)

        mv = lax.fori_loop(0, _CHUNK_A // 16, grp, mv)
        pltpu.async_copy(svb[b], scores_hbm.at[pl.ds(off, _CHUNK_A)], semw)
        return mv

    fire(0, 0)

    def body(k2, mv):
        for b in (0, 1):
            cur = 2 * k2 + b
            fire(cur + 1, 1 - b)
            wait_gathers(b)
            mv = compute(cur, b, mv)
        return mv

    mv = lax.fori_loop(0, (nch - 1) // 2, body,
                       jnp.full((16,), -3.0e38, jnp.float32))
    wait_gathers(0)
    mv = compute(nch - 1, 0, mv)
    pltpu.make_async_copy(sv0, scores_hbm.at[pl.ds(0, _CHUNK_A)], semw).wait()
    pltpu.make_async_copy(sv1, scores_hbm.at[pl.ds(0, _CHUNK_A)], semw).wait()
    mx_v[...] = mv
    pltpu.sync_copy(mx_v, pmax_hbm.at[pl.ds(wid * 16, 16)])


def _sc_scores(q, k, edge_index):
    mesh = plsc.VectorSubcoreMesh(core_axis_name="c", subcore_axis_name="s",
                                  num_cores=NC, num_subcores=NS)
    kern = pl.kernel(
        _score_body,
        out_type=[
            jax.ShapeDtypeStruct((E,), jnp.float32),
            jax.ShapeDtypeStruct((NT * 16,), jnp.float32),
        ],
        mesh=mesh,
        compiler_params=pltpu.CompilerParams(use_tc_tiling_on_sc=False, needs_layout_passes=False),
        scratch_types=[
            pltpu.VMEM((2, _CHUNK_A), jnp.int32),
            pltpu.VMEM((2, _CHUNK_A), jnp.int32),
            pltpu.VMEM((_CHUNK_A, H), jnp.float32),
            pltpu.VMEM((_CHUNK_A, H), jnp.float32),
            pltpu.VMEM((_CHUNK_A, H), jnp.float32),
            pltpu.VMEM((_CHUNK_A, H), jnp.float32),
            pltpu.VMEM((_CHUNK_A,), jnp.float32),
            pltpu.VMEM((_CHUNK_A,), jnp.float32),
            pltpu.VMEM((16,), jnp.float32),
            pltpu.SemaphoreType.DMA,
            pltpu.SemaphoreType.DMA,
            pltpu.SemaphoreType.DMA,
        ],
    )
    return kern(q, k, edge_index)


def _agg_body(scores_hbm, pmax_hbm, ei_hbm, vflat_hbm, out_hbm,
              ei0, ei1, dst0, dst1, ix0, ix1, ev0, ev1, vr0, vr1,
              sc_v, arows, den_v, pm_v, agg_sh, den_sh,
              semg0, semg1, sems):
    c = lax.axis_index("c")
    t = lax.axis_index("s")
    eib = (ei0, ei1)
    dstb = (dst0, dst1)
    ixb = (ix0, ix1)
    evb = (ev0, ev1)
    vrb = (vr0, vr1)
    semg = (semg0, semg1)

    def zrow(i, _):
        for j in range(_HH // 16):
            arows[i, pl.ds(j * 16, 16)] = jnp.zeros((16,), jnp.float32)
        return 0

    lax.fori_loop(0, _RCH, zrow, 0)

    def zev(g, _):
        ev0[pl.ds(g * 16, 16)] = jnp.zeros((16,), jnp.float32)
        return 0

    lax.fori_loop(0, _CHUNK_B // 16, zev, 0)

    def zagg(rc, _):
        pltpu.sync_copy(arows, agg_sh.at[pl.ds(t * _RPT + rc * _RCH, _RCH)])
        return 0

    lax.fori_loop(0, _RPT // _RCH, zagg, 0)

    def zden(zi, _):
        pltpu.sync_copy(ev0, den_sh.at[pl.ds(t * _RPT + zi * _CHUNK_B, _CHUNK_B)])
        return 0

    lax.fori_loop(0, _RPT // _CHUNK_B, zden, 0)
    plsc.subcore_barrier()

    pltpu.sync_copy(pmax_hbm, pm_v)
    mv = pm_v[pl.ds(0, 16)]
    for g in range(1, NT):
        mv = jnp.maximum(mv, pm_v[pl.ds(g * 16, 16)])
    gmax = jnp.max(mv)

    ebase = t * _EPT_B
    coff = jnp.full((16,), c * N, jnp.int32)
    nch = _EPT_B // _CHUNK_B

    def prep(cur, b):
        off = ebase + cur * _CHUNK_B
        pltpu.sync_copy(ei_hbm.at[:, pl.ds(off, _CHUNK_B)], eib[b])
        pltpu.sync_copy(scores_hbm.at[pl.ds(off, _CHUNK_B)], sc_v)

        def vec(g, _):
            sl = pl.ds(g * 16, 16)
            evb[b][sl] = jnp.exp(sc_v[sl] - gmax)
            ixb[b][sl] = eib[b][0, sl] + coff
            dstb[b][sl] = eib[b][1, sl]
            return 0

        lax.fori_loop(0, _CHUNK_B // 16, vec, 0)

    def fire_gather(b):
        pltpu.async_copy(vflat_hbm.at[ixb[b]], vrb[b], semg[b])

    def wait_gather(b):
        pltpu.make_async_copy(vflat_hbm.at[pl.ds(0, _CHUNK_B)], vrb[b],
                              semg[b]).wait()

    def wait_scatter_pair():
        pltpu.make_async_copy(vr0, agg_sh.at[pl.ds(0, _CHUNK_B)], sems).wait()
        pltpu.make_async_copy(ev0, den_sh.at[pl.ds(0, _CHUNK_B)], sems).wait()

    def process(b):
        wait_gather(b)

        def edge(e, _):
            eb = plsc.load_gather(evb[b], [jnp.full((16,), e, jnp.int32)])
            for j in range(_HH // 16):
                sl = pl.ds(j * 16, 16)
                vrb[b][e, sl] = vrb[b][e, sl] * eb
            return 0

        lax.fori_loop(0, _CHUNK_B, edge, 0)
        pltpu.async_copy(vrb[b], agg_sh.at[dstb[b]], sems, add=True)
        pltpu.async_copy(evb[b], den_sh.at[dstb[b]], sems, add=True)

    prep(0, 0)
    fire_gather(0)

    def body(k2, _):
        for b2 in (0, 1):
            ci = 2 * k2 + b2

            @pl.when(ci >= 1)
            def _():
                wait_scatter_pair()

            prep(ci + 1, 1 - b2)
            fire_gather(1 - b2)
            process(b2)
        return 0

    lax.fori_loop(0, (nch - 2) // 2, body, 0)
    wait_scatter_pair()
    prep(nch - 1, 1)
    fire_gather(1)
    process(0)
    wait_scatter_pair()
    process(1)
    wait_scatter_pair()
    plsc.subcore_barrier()

    rbase = t * _RPT

    def nchunk(rc, _):
        r0 = rbase + rc * _RCH
        pltpu.sync_copy(agg_sh.at[pl.ds(r0, _RCH)], arows)
        pltpu.sync_copy(den_sh.at[pl.ds(r0, _RCH)], den_v)

        def row(r, _):
            db = plsc.load_gather(den_v, [jnp.full((16,), r, jnp.int32)]) + 1e-16
            for j in range(_HH // 16):
                sl = pl.ds(j * 16, 16)
                arows[r, sl] = arows[r, sl] / db
            return 0

        lax.fori_loop(0, _RCH, row, 0)
        pltpu.sync_copy(arows, out_hbm.at[pl.ds(c * NP + r0, _RCH)])
        return 0

    lax.fori_loop(0, _RPT // _RCH, nchunk, 0)


def _sc_aggregate(scores, pmax, edge_index, vflat):
    mesh = plsc.VectorSubcoreMesh(core_axis_name="c", subcore_axis_name="s",
                                  num_cores=NC, num_subcores=NS)
    kern = pl.kernel(
        _agg_body,
        out_type=jax.ShapeDtypeStruct((2 * NP, _HH), jnp.float32),
        mesh=mesh,
        compiler_params=pltpu.CompilerParams(use_tc_tiling_on_sc=False, needs_layout_passes=False),
        scratch_types=[
            pltpu.VMEM((2, _CHUNK_B), jnp.int32),
            pltpu.VMEM((2, _CHUNK_B), jnp.int32),
            pltpu.VMEM((_CHUNK_B,), jnp.int32),
            pltpu.VMEM((_CHUNK_B,), jnp.int32),
            pltpu.VMEM((_CHUNK_B,), jnp.int32),
            pltpu.VMEM((_CHUNK_B,), jnp.int32),
            pltpu.VMEM((_CHUNK_B,), jnp.float32),
            pltpu.VMEM((_CHUNK_B,), jnp.float32),
            pltpu.VMEM((_CHUNK_B, _HH), jnp.float32),
            pltpu.VMEM((_CHUNK_B, _HH), jnp.float32),
            pltpu.VMEM((_CHUNK_B,), jnp.float32),
            pltpu.VMEM((_RCH, _HH), jnp.float32),
            pltpu.VMEM((_RCH,), jnp.float32),
            pltpu.VMEM((NT * 16,), jnp.float32),
            pltpu.VMEM_SHARED((NP, _HH), jnp.float32),
            pltpu.VMEM_SHARED((NP,), jnp.float32),
            pltpu.SemaphoreType.DMA,
            pltpu.SemaphoreType.DMA,
            pltpu.SemaphoreType.DMA,
        ],
    )
    return kern(scores, pmax, edge_index, vflat)


def kernel(x, edge_index, W_emb, b_emb, Wq, bq, Wk, bk, Wv, bv, Ws, bs):
    h = _matmul_bias(x, W_emb, b_emb)

    wcat = jnp.concatenate([Wq[1:], Wk[1:], Wv[1:], Ws[1:]], axis=1)
    ts = np.linspace(0.0, 1.0, N_STEPS).astype(np.float32)

    ys = [h]
    y = h
    for i in range(N_STEPS - 1):
        tcur = float(ts[i])
        dt = float(ts[i + 1] - ts[i])
        bcat = jnp.concatenate(
            [bq + tcur * Wq[0], bk + tcur * Wk[0],
             bv + tcur * Wv[0], bs + tcur * Ws[0]]
        )
        q, k, v0, v1, s = _qkvs(y, wcat, bcat)
        vflat = jnp.concatenate([v0, v1], axis=0)
        scores, pmax = _sc_scores(q, k, edge_index)
        aggflat = _sc_aggregate(scores, pmax, edge_index, vflat)
        aggn = jnp.concatenate([aggflat[:N], aggflat[NP:NP + N]], axis=1)
        y = _euler_update(y, aggn, s, dt)
        ys.append(y)
    return jnp.stack(ys, axis=0)


# speedup vs baseline: 8.4646x; 3.4437x over previous
import jax
import jax.numpy as jnp
import numpy as np
from jax import lax
from jax.experimental import pallas as pl
from jax.experimental.pallas import tpu as pltpu
from jax.experimental.pallas import tpu_sc as plsc

N = 10000
E = 320000
D_IN = 128
H = 256
N_STEPS = 4
NP = 10240
NC, NS = 2, 16
NT = NC * NS
SCALE = 0.0625


def _matmul_bias(xa, w, b, bm=2000):
    n, kd = xa.shape
    m = w.shape[1]

    def body(x_ref, w_ref, b_ref, o_ref):
        o_ref[...] = (
            jnp.dot(x_ref[...], w_ref[...], preferred_element_type=jnp.float32)
            + b_ref[...]
        )

    return pl.pallas_call(
        body,
        grid=(n // bm,),
        in_specs=[
            pl.BlockSpec((bm, kd), lambda i: (i, 0)),
            pl.BlockSpec((kd, m), lambda i: (0, 0)),
            pl.BlockSpec((1, m), lambda i: (0, 0)),
        ],
        out_specs=pl.BlockSpec((bm, m), lambda i: (i, 0)),
        out_shape=jax.ShapeDtypeStruct((n, m), jnp.float32),
    )(xa, w, b.reshape(1, m))


def _qkvs(y, wcat, bcat, bm=2000):
    def body(y_ref, w_ref, b_ref, q_ref, k_ref, v0_ref, v1_ref, s_ref):
        acc = (
            jnp.dot(y_ref[...], w_ref[...], preferred_element_type=jnp.float32)
            + b_ref[...]
        )
        q_ref[...] = acc[:, 0:256]
        k_ref[...] = acc[:, 256:512]
        v0_ref[...] = acc[:, 512:640]
        v1_ref[...] = acc[:, 640:768]
        s_ref[...] = acc[:, 768:1024]

    grid = (N // bm,)
    return pl.pallas_call(
        body,
        grid=grid,
        in_specs=[
            pl.BlockSpec((bm, H), lambda i: (i, 0)),
            pl.BlockSpec((H, 4 * H), lambda i: (0, 0)),
            pl.BlockSpec((1, 4 * H), lambda i: (0, 0)),
        ],
        out_specs=[
            pl.BlockSpec((bm, H), lambda i: (i, 0)),
            pl.BlockSpec((bm, H), lambda i: (i, 0)),
            pl.BlockSpec((bm, H // 2), lambda i: (i, 0)),
            pl.BlockSpec((bm, H // 2), lambda i: (i, 0)),
            pl.BlockSpec((bm, H), lambda i: (i, 0)),
        ],
        out_shape=[
            jax.ShapeDtypeStruct((N, H), jnp.float32),
            jax.ShapeDtypeStruct((N, H), jnp.float32),
            jax.ShapeDtypeStruct((N, H // 2), jnp.float32),
            jax.ShapeDtypeStruct((N, H // 2), jnp.float32),
            jax.ShapeDtypeStruct((N, H), jnp.float32),
        ],
    )(y, wcat, bcat.reshape(1, 4 * H))


def _euler_update(y, aggn, s, dt, bm=2000):
    def body(y_ref, a_ref, s_ref, o_ref):
        o_ref[...] = y_ref[...] + dt * (a_ref[...] + s_ref[...])

    return pl.pallas_call(
        body,
        grid=(N // bm,),
        in_specs=[pl.BlockSpec((bm, H), lambda i: (i, 0))] * 3,
        out_specs=pl.BlockSpec((bm, H), lambda i: (i, 0)),
        out_shape=jax.ShapeDtypeStruct((N, H), jnp.float32),
    )(y, aggn, s)


_CHUNK_A = 80
_EPT_A = E // NT
_CHUNK_B = 80
_EPT_B = E // NS
_RPT = NP // NS
_RCH = 64
_HH = H // 2


def _score_body(q_hbm, k_hbm, ei_hbm, scores_hbm, pmax_hbm,
                ei0, ei1, qr0, qr1, kr0, kr1, sv0, sv1, mx_v,
                sem0, sem1, semw):
    wid = lax.axis_index("s") * NC + lax.axis_index("c")
    base = wid * _EPT_A
    lanes = lax.iota(jnp.int32, 16)
    ones = jnp.full((16,), 1, jnp.int32)
    eib = (ei0, ei1)
    qrb = (qr0, qr1)
    krb = (kr0, kr1)
    svb = (sv0, sv1)
    semb = (sem0, sem1)
    nch = _EPT_A // _CHUNK_A

    def fire(cur, b):
        off = base + cur * _CHUNK_A
        pltpu.sync_copy(ei_hbm.at[:, pl.ds(off, _CHUNK_A)], eib[b])
        pltpu.async_copy(q_hbm.at[eib[b].at[1]], qrb[b], semb[b])
        pltpu.async_copy(k_hbm.at[eib[b].at[0]], krb[b], semb[b])

    def wait_gathers(b):
        pltpu.make_async_copy(q_hbm.at[pl.ds(0, _CHUNK_A)], qrb[b], semb[b]).wait()
        pltpu.make_async_copy(k_hbm.at[pl.ds(0, _CHUNK_A)], krb[b], semb[b]).wait()

    def compute(cur, b, mv):
        off = base + cur * _CHUNK_A

        @pl.when(cur >= 2)
        def _():
            pltpu.make_async_copy(svb[b], scores_hbm.at[pl.ds(0, _CHUNK_A)],
                                  semw).wait()

        def grp(g, mcur):
            eidx = lanes + g * 16
            zf = jnp.zeros((16,), jnp.float32)

            def jblk(jb, carry):
                cidx, a0, a1, a2, a3 = carry
                accs = [a0, a1, a2, a3]
                for u in range(16):
                    qv = plsc.load_gather(qrb[b], [eidx, cidx])
                    kv = plsc.load_gather(krb[b], [eidx, cidx])
                    accs[u % 4] = accs[u % 4] + qv * kv
                    cidx = cidx + ones
                return (cidx, accs[0], accs[1], accs[2], accs[3])

            cidx, a0, a1, a2, a3 = lax.fori_loop(
                0, 15, jblk, (lanes, zf, zf, zf, zf))
            accs = [a0, a1, a2, a3]
            cmod = jnp.full((16,), H, jnp.int32)
            for u in range(16):
                cw = jnp.where(cidx >= cmod, cidx - cmod, cidx)
                qv = plsc.load_gather(qrb[b], [eidx, cw])
                kv = plsc.load_gather(krb[b], [eidx, cw])
                accs[u % 4] = accs[u % 4] + qv * kv
                cidx = cidx + ones
            sc = ((accs[0] + accs[1]) + (accs[2] + accs[3])) * SCALE
            svb[b][pl.ds(g * 16, 16)] = sc
            return jnp.maximum(mcur, sc)

        mv = lax.fori_loop(0, _CHUNK_A // 16, grp, mv)
        pltpu.async_copy(svb[b], scores_hbm.at[pl.ds(off, _CHUNK_A)], semw)
        return mv

    fire(0, 0)

    def body(k2, mv):
        for b in (0, 1):
            cur = 2 * k2 + b
            fire(cur + 1, 1 - b)
            wait_gathers(b)
            mv = compute(cur, b, mv)
        return mv

    mv = lax.fori_loop(0, (nch - 1) // 2, body,
                       jnp.full((16,), -3.0e38, jnp.float32))
    wait_gathers(0)
    mv = compute(nch - 1, 0, mv)
    pltpu.make_async_copy(sv0, scores_hbm.at[pl.ds(0, _CHUNK_A)], semw).wait()
    pltpu.make_async_copy(sv1, scores_hbm.at[pl.ds(0, _CHUNK_A)], semw).wait()
    mx_v[...] = mv
    pltpu.sync_copy(mx_v, pmax_hbm.at[pl.ds(wid * 16, 16)])


def _sc_scores(q, k, edge_index):
    mesh = plsc.VectorSubcoreMesh(core_axis_name="c", subcore_axis_name="s",
                                  num_cores=NC, num_subcores=NS)
    kern = pl.kernel(
        _score_body,
        out_type=[
            jax.ShapeDtypeStruct((E,), jnp.float32),
            jax.ShapeDtypeStruct((NT * 16,), jnp.float32),
        ],
        mesh=mesh,
        compiler_params=pltpu.CompilerParams(use_tc_tiling_on_sc=False, needs_layout_passes=False),
        scratch_types=[
            pltpu.VMEM((2, _CHUNK_A), jnp.int32),
            pltpu.VMEM((2, _CHUNK_A), jnp.int32),
            pltpu.VMEM((_CHUNK_A, H), jnp.float32),
            pltpu.VMEM((_CHUNK_A, H), jnp.float32),
            pltpu.VMEM((_CHUNK_A, H), jnp.float32),
            pltpu.VMEM((_CHUNK_A, H), jnp.float32),
            pltpu.VMEM((_CHUNK_A,), jnp.float32),
            pltpu.VMEM((_CHUNK_A,), jnp.float32),
            pltpu.VMEM((16,), jnp.float32),
            pltpu.SemaphoreType.DMA,
            pltpu.SemaphoreType.DMA,
            pltpu.SemaphoreType.DMA,
        ],
    )
    return kern(q, k, edge_index)


def _agg_body(scores_hbm, pmax_hbm, ei_hbm, vflat_hbm, out_hbm,
              ei0, ei1, dst0, dst1, ix0, ix1, ev0, ev1, vr0, vr1,
              sc_v, arows, den_v, pm_v, agg_sh, den_sh,
              semg0, semg1, sems):
    c = lax.axis_index("c")
    t = lax.axis_index("s")
    eib = (ei0, ei1)
    dstb = (dst0, dst1)
    ixb = (ix0, ix1)
    evb = (ev0, ev1)
    vrb = (vr0, vr1)
    semg = (semg0, semg1)

    def zrow(i, _):
        for j in range(_HH // 16):
            arows[i, pl.ds(j * 16, 16)] = jnp.zeros((16,), jnp.float32)
        return 0

    lax.fori_loop(0, _RCH, zrow, 0)

    def zev(g, _):
        ev0[pl.ds(g * 16, 16)] = jnp.zeros((16,), jnp.float32)
        return 0

    lax.fori_loop(0, _CHUNK_B // 16, zev, 0)

    def zagg(rc, _):
        pltpu.sync_copy(arows, agg_sh.at[pl.ds(t * _RPT + rc * _RCH, _RCH)])
        return 0

    lax.fori_loop(0, _RPT // _RCH, zagg, 0)

    def zden(zi, _):
        pltpu.sync_copy(ev0, den_sh.at[pl.ds(t * _RPT + zi * _CHUNK_B, _CHUNK_B)])
        return 0

    lax.fori_loop(0, _RPT // _CHUNK_B, zden, 0)
    plsc.subcore_barrier()

    pltpu.sync_copy(pmax_hbm, pm_v)
    mv = pm_v[pl.ds(0, 16)]
    for g in range(1, NT):
        mv = jnp.maximum(mv, pm_v[pl.ds(g * 16, 16)])
    gmax = jnp.max(mv)

    ebase = t * _EPT_B
    coff = jnp.full((16,), c * N, jnp.int32)
    nch = _EPT_B // _CHUNK_B

    def prep(cur, b):
        off = ebase + cur * _CHUNK_B
        pltpu.sync_copy(ei_hbm.at[:, pl.ds(off, _CHUNK_B)], eib[b])
        pltpu.sync_copy(scores_hbm.at[pl.ds(off, _CHUNK_B)], sc_v)

        def vec(g, _):
            sl = pl.ds(g * 16, 16)
            evb[b][sl] = jnp.exp(sc_v[sl] - gmax)
            ixb[b][sl] = eib[b][0, sl] + coff
            dstb[b][sl] = eib[b][1, sl]
            return 0

        lax.fori_loop(0, _CHUNK_B // 16, vec, 0)

    def fire_gather(b):
        pltpu.async_copy(vflat_hbm.at[ixb[b]], vrb[b], semg[b])

    def wait_gather(b):
        pltpu.make_async_copy(vflat_hbm.at[pl.ds(0, _CHUNK_B)], vrb[b],
                              semg[b]).wait()

    def wait_scatter_pair():
        pltpu.make_async_copy(vr0, agg_sh.at[pl.ds(0, _CHUNK_B)], sems).wait()
        pltpu.make_async_copy(ev0, den_sh.at[pl.ds(0, _CHUNK_B)], sems).wait()

    def process(b):
        wait_gather(b)

        def edge(e, _):
            eb = plsc.load_gather(evb[b], [jnp.full((16,), e, jnp.int32)])
            for j in range(_HH // 16):
                sl = pl.ds(j * 16, 16)
                vrb[b][e, sl] = vrb[b][e, sl] * eb
            return 0

        lax.fori_loop(0, _CHUNK_B, edge, 0)
        pltpu.async_copy(vrb[b], agg_sh.at[dstb[b]], sems, add=True)
        pltpu.async_copy(evb[b], den_sh.at[dstb[b]], sems, add=True)

    prep(0, 0)
    fire_gather(0)

    def body(k2, _):
        for b2 in (0, 1):
            ci = 2 * k2 + b2

            @pl.when(ci >= 1)
            def _():
                wait_scatter_pair()

            prep(ci + 1, 1 - b2)
            fire_gather(1 - b2)
            process(b2)
        return 0

    lax.fori_loop(0, (nch - 2) // 2, body, 0)
    wait_scatter_pair()
    prep(nch - 1, 1)
    fire_gather(1)
    process(0)
    wait_scatter_pair()
    process(1)
    wait_scatter_pair()
    plsc.subcore_barrier()

    rbase = t * _RPT

    def nchunk(rc, _):
        r0 = rbase + rc * _RCH
        pltpu.sync_copy(agg_sh.at[pl.ds(r0, _RCH)], arows)
        pltpu.sync_copy(den_sh.at[pl.ds(r0, _RCH)], den_v)

        def row(r, _):
            db = plsc.load_gather(den_v, [jnp.full((16,), r, jnp.int32)]) + 1e-16
            for j in range(_HH // 16):
                sl = pl.ds(j * 16, 16)
                arows[r, sl] = arows[r, sl] / db
            return 0

        lax.fori_loop(0, _RCH, row, 0)
        pltpu.sync_copy(arows, out_hbm.at[pl.ds(c * NP + r0, _RCH)])
        return 0

    lax.fori_loop(0, _RPT // _RCH, nchunk, 0)


def _sc_aggregate(scores, pmax, edge_index, vflat):
    mesh = plsc.VectorSubcoreMesh(core_axis_name="c", subcore_axis_name="s",
                                  num_cores=NC, num_subcores=NS)
    kern = pl.kernel(
        _agg_body,
        out_type=jax.ShapeDtypeStruct((2 * NP, _HH), jnp.float32),
        mesh=mesh,
        compiler_params=pltpu.CompilerParams(use_tc_tiling_on_sc=False, needs_layout_passes=False),
        scratch_types=[
            pltpu.VMEM((2, _CHUNK_B), jnp.int32),
            pltpu.VMEM((2, _CHUNK_B), jnp.int32),
            pltpu.VMEM((_CHUNK_B,), jnp.int32),
            pltpu.VMEM((_CHUNK_B,), jnp.int32),
            pltpu.VMEM((_CHUNK_B,), jnp.int32),
            pltpu.VMEM((_CHUNK_B,), jnp.int32),
            pltpu.VMEM((_CHUNK_B,), jnp.float32),
            pltpu.VMEM((_CHUNK_B,), jnp.float32),
            pltpu.VMEM((_CHUNK_B, _HH), jnp.float32),
            pltpu.VMEM((_CHUNK_B, _HH), jnp.float32),
            pltpu.VMEM((_CHUNK_B,), jnp.float32),
            pltpu.VMEM((_RCH, _HH), jnp.float32),
            pltpu.VMEM((_RCH,), jnp.float32),
            pltpu.VMEM((NT * 16,), jnp.float32),
            pltpu.VMEM_SHARED((NP, _HH), jnp.float32),
            pltpu.VMEM_SHARED((NP,), jnp.float32),
            pltpu.SemaphoreType.DMA,
            pltpu.SemaphoreType.DMA,
            pltpu.SemaphoreType.DMA,
        ],
    )
    return kern(scores, pmax, edge_index, vflat)


def kernel(x, edge_index, W_emb, b_emb, Wq, bq, Wk, bk, Wv, bv, Ws, bs):
    h = _matmul_bias(x, W_emb, b_emb)

    wcat = jnp.concatenate([Wq[1:], Wk[1:], Wv[1:], Ws[1:]], axis=1)
    ts = np.linspace(0.0, 1.0, N_STEPS).astype(np.float32)

    ys = [h]
    y = h
    for i in range(N_STEPS - 1):
        tcur = float(ts[i])
        dt = float(ts[i + 1] - ts[i])
        bcat = jnp.concatenate(
            [bq + tcur * Wq[0], bk + tcur * Wk[0],
             bv + tcur * Wv[0], bs + tcur * Ws[0]]
        )
        q, k, v0, v1, s = _qkvs(y, wcat, bcat)
        vflat = jnp.concatenate([v0, v1], axis=0)
        scores, pmax = _sc_scores(q, k, edge_index)
        aggflat = _sc_aggregate(scores, pmax, edge_index, vflat)
        aggn = jnp.concatenate([aggflat[:N], aggflat[NP:NP + N]], axis=1)
        y = _euler_update(y, aggn, s, dt)
        ys.append(y)
    return jnp.stack(ys, axis=0)


# speedup vs baseline: 11.0077x; 1.3004x over previous
import jax
import jax.numpy as jnp
import numpy as np
from jax import lax
from jax.experimental import pallas as pl
from jax.experimental.pallas import tpu as pltpu
from jax.experimental.pallas import tpu_sc as plsc

N = 10000
E = 320000
D_IN = 128
H = 256
N_STEPS = 4
NP = 10240
NC, NS = 2, 16
NT = NC * NS
SCALE = 0.0625


def _matmul_bias(xa, w, b, bm=2000):
    n, kd = xa.shape
    m = w.shape[1]

    def body(x_ref, w_ref, b_ref, o_ref):
        o_ref[...] = (
            jnp.dot(x_ref[...], w_ref[...], preferred_element_type=jnp.float32)
            + b_ref[...]
        )

    return pl.pallas_call(
        body,
        grid=(n // bm,),
        in_specs=[
            pl.BlockSpec((bm, kd), lambda i: (i, 0)),
            pl.BlockSpec((kd, m), lambda i: (0, 0)),
            pl.BlockSpec((1, m), lambda i: (0, 0)),
        ],
        out_specs=pl.BlockSpec((bm, m), lambda i: (i, 0)),
        out_shape=jax.ShapeDtypeStruct((n, m), jnp.float32),
    )(xa, w, b.reshape(1, m))


def _qkvs(y, wcat, bcat, bm=2000):
    def body(y_ref, w_ref, b_ref, q_ref, k_ref, v0_ref, v1_ref, s_ref):
        acc = (
            jnp.dot(y_ref[...], w_ref[...], preferred_element_type=jnp.float32)
            + b_ref[...]
        )
        q_ref[...] = acc[:, 0:256]
        k_ref[...] = acc[:, 256:512]
        v0_ref[...] = acc[:, 512:640]
        v1_ref[...] = acc[:, 640:768]
        s_ref[...] = acc[:, 768:1024]

    grid = (N // bm,)
    return pl.pallas_call(
        body,
        grid=grid,
        in_specs=[
            pl.BlockSpec((bm, H), lambda i: (i, 0)),
            pl.BlockSpec((H, 4 * H), lambda i: (0, 0)),
            pl.BlockSpec((1, 4 * H), lambda i: (0, 0)),
        ],
        out_specs=[
            pl.BlockSpec((bm, H), lambda i: (i, 0)),
            pl.BlockSpec((bm, H), lambda i: (i, 0)),
            pl.BlockSpec((bm, H // 2), lambda i: (i, 0)),
            pl.BlockSpec((bm, H // 2), lambda i: (i, 0)),
            pl.BlockSpec((bm, H), lambda i: (i, 0)),
        ],
        out_shape=[
            jax.ShapeDtypeStruct((N, H), jnp.float32),
            jax.ShapeDtypeStruct((N, H), jnp.float32),
            jax.ShapeDtypeStruct((N, H // 2), jnp.float32),
            jax.ShapeDtypeStruct((N, H // 2), jnp.float32),
            jax.ShapeDtypeStruct((N, H), jnp.float32),
        ],
    )(y, wcat, bcat.reshape(1, 4 * H))


def _euler_update(y, aggn, s, dt, bm=2000):
    def body(y_ref, a_ref, s_ref, o_ref):
        o_ref[...] = y_ref[...] + dt * (a_ref[...] + s_ref[...])

    return pl.pallas_call(
        body,
        grid=(N // bm,),
        in_specs=[pl.BlockSpec((bm, H), lambda i: (i, 0))] * 3,
        out_specs=pl.BlockSpec((bm, H), lambda i: (i, 0)),
        out_shape=jax.ShapeDtypeStruct((N, H), jnp.float32),
    )(y, aggn, s)


_CHUNK_A = 80
_EPT_A = E // NT
_CHUNK_B = 80
_EPT_B = E // NS
_RPT = NP // NS
_RCH = 64
_HH = H // 2


def _score_body(q_hbm, k_hbm, ei_hbm, scores_hbm, pmax_hbm,
                ei0, ei1, qr0, qr1, kr0, kr1, sv0, sv1, mx_v,
                sem0, sem1, semi0, semi1, semw):
    wid = lax.axis_index("s") * NC + lax.axis_index("c")
    base = wid * _EPT_A
    lanes = lax.iota(jnp.int32, 16)
    ones = jnp.full((16,), 1, jnp.int32)
    eib = (ei0, ei1)
    qrb = (qr0, qr1)
    krb = (kr0, kr1)
    svb = (sv0, sv1)
    semb = (sem0, sem1)
    semi = (semi0, semi1)
    nch = _EPT_A // _CHUNK_A

    def fire_idx(cur, b):
        off = base + cur * _CHUNK_A
        pltpu.async_copy(ei_hbm.at[:, pl.ds(off, _CHUNK_A)], eib[b], semi[b])

    def wait_idx(b):
        pltpu.make_async_copy(ei_hbm.at[:, pl.ds(0, _CHUNK_A)], eib[b],
                              semi[b]).wait()

    def fire_gathers(b):
        pltpu.async_copy(q_hbm.at[eib[b].at[1]], qrb[b], semb[b])
        pltpu.async_copy(k_hbm.at[eib[b].at[0]], krb[b], semb[b])

    def wait_gathers(b):
        pltpu.make_async_copy(q_hbm.at[pl.ds(0, _CHUNK_A)], qrb[b], semb[b]).wait()
        pltpu.make_async_copy(k_hbm.at[pl.ds(0, _CHUNK_A)], krb[b], semb[b]).wait()

    def compute(cur, b, mv):
        off = base + cur * _CHUNK_A

        @pl.when(cur >= 2)
        def _():
            pltpu.make_async_copy(svb[b], scores_hbm.at[pl.ds(0, _CHUNK_A)],
                                  semw).wait()

        def grp(g, mcur):
            eidx = lanes + g * 16
            zf = jnp.zeros((16,), jnp.float32)

            def jblk(jb, carry):
                cidx, a0, a1, a2, a3 = carry
                accs = [a0, a1, a2, a3]
                for u in range(16):
                    qv = plsc.load_gather(qrb[b], [eidx, cidx])
                    kv = plsc.load_gather(krb[b], [eidx, cidx])
                    accs[u % 4] = accs[u % 4] + qv * kv
                    cidx = cidx + ones
                return (cidx, accs[0], accs[1], accs[2], accs[3])

            cidx, a0, a1, a2, a3 = lax.fori_loop(
                0, 15, jblk, (lanes, zf, zf, zf, zf))
            accs = [a0, a1, a2, a3]
            cmod = jnp.full((16,), H, jnp.int32)
            for u in range(16):
                cw = jnp.where(cidx >= cmod, cidx - cmod, cidx)
                qv = plsc.load_gather(qrb[b], [eidx, cw])
                kv = plsc.load_gather(krb[b], [eidx, cw])
                accs[u % 4] = accs[u % 4] + qv * kv
                cidx = cidx + ones
            sc = ((accs[0] + accs[1]) + (accs[2] + accs[3])) * SCALE
            svb[b][pl.ds(g * 16, 16)] = sc
            return jnp.maximum(mcur, sc)

        mv = lax.fori_loop(0, _CHUNK_A // 16, grp, mv)
        pltpu.async_copy(svb[b], scores_hbm.at[pl.ds(off, _CHUNK_A)], semw)
        return mv

    fire_idx(0, 0)
    fire_idx(1, 1)
    wait_idx(0)
    fire_gathers(0)

    def body(k2, mv):
        for b in (0, 1):
            cur = 2 * k2 + b
            wait_gathers(b)

            @pl.when(cur + 2 < nch)
            def _():
                fire_idx(cur + 2, b)

            wait_idx(1 - b)
            fire_gathers(1 - b)
            mv = compute(cur, b, mv)
        return mv

    mv = lax.fori_loop(0, (nch - 1) // 2, body,
                       jnp.full((16,), -3.0e38, jnp.float32))
    wait_gathers(0)
    mv = compute(nch - 1, 0, mv)
    pltpu.make_async_copy(sv0, scores_hbm.at[pl.ds(0, _CHUNK_A)], semw).wait()
    pltpu.make_async_copy(sv1, scores_hbm.at[pl.ds(0, _CHUNK_A)], semw).wait()
    mx_v[...] = mv
    pltpu.sync_copy(mx_v, pmax_hbm.at[pl.ds(wid * 16, 16)])


def _sc_scores(q, k, edge_index):
    mesh = plsc.VectorSubcoreMesh(core_axis_name="c", subcore_axis_name="s",
                                  num_cores=NC, num_subcores=NS)
    kern = pl.kernel(
        _score_body,
        out_type=[
            jax.ShapeDtypeStruct((E,), jnp.float32),
            jax.ShapeDtypeStruct((NT * 16,), jnp.float32),
        ],
        mesh=mesh,
        compiler_params=pltpu.CompilerParams(use_tc_tiling_on_sc=False, needs_layout_passes=False),
        scratch_types=[
            pltpu.VMEM((2, _CHUNK_A), jnp.int32),
            pltpu.VMEM((2, _CHUNK_A), jnp.int32),
            pltpu.VMEM((_CHUNK_A, H), jnp.float32),
            pltpu.VMEM((_CHUNK_A, H), jnp.float32),
            pltpu.VMEM((_CHUNK_A, H), jnp.float32),
            pltpu.VMEM((_CHUNK_A, H), jnp.float32),
            pltpu.VMEM((_CHUNK_A,), jnp.float32),
            pltpu.VMEM((_CHUNK_A,), jnp.float32),
            pltpu.VMEM((16,), jnp.float32),
            pltpu.SemaphoreType.DMA,
            pltpu.SemaphoreType.DMA,
            pltpu.SemaphoreType.DMA,
            pltpu.SemaphoreType.DMA,
            pltpu.SemaphoreType.DMA,
        ],
    )
    return kern(q, k, edge_index)


def _agg_body(scores_hbm, pmax_hbm, ei_hbm, vflat_hbm, out_hbm,
              ei0, ei1, dst0, dst1, ix0, ix1, ev0, ev1, vr0, vr1,
              sc0, sc1, arows, den_v, pm_v, agg_sh, den_sh,
              semg0, semg1, semi0, semi1, sems):
    c = lax.axis_index("c")
    t = lax.axis_index("s")
    eib = (ei0, ei1)
    dstb = (dst0, dst1)
    ixb = (ix0, ix1)
    evb = (ev0, ev1)
    vrb = (vr0, vr1)
    scb = (sc0, sc1)
    semg = (semg0, semg1)
    semi = (semi0, semi1)

    def zrow(i, _):
        for j in range(_HH // 16):
            arows[i, pl.ds(j * 16, 16)] = jnp.zeros((16,), jnp.float32)
        return 0

    lax.fori_loop(0, _RCH, zrow, 0)

    def zev(g, _):
        ev0[pl.ds(g * 16, 16)] = jnp.zeros((16,), jnp.float32)
        return 0

    lax.fori_loop(0, _CHUNK_B // 16, zev, 0)

    def zagg(rc, _):
        pltpu.sync_copy(arows, agg_sh.at[pl.ds(t * _RPT + rc * _RCH, _RCH)])
        return 0

    lax.fori_loop(0, _RPT // _RCH, zagg, 0)

    def zden(zi, _):
        pltpu.sync_copy(ev0, den_sh.at[pl.ds(t * _RPT + zi * _CHUNK_B, _CHUNK_B)])
        return 0

    lax.fori_loop(0, _RPT // _CHUNK_B, zden, 0)
    plsc.subcore_barrier()

    pltpu.sync_copy(pmax_hbm, pm_v)
    mv = pm_v[pl.ds(0, 16)]
    for g in range(1, NT):
        mv = jnp.maximum(mv, pm_v[pl.ds(g * 16, 16)])
    gmax = jnp.max(mv)

    ebase = t * _EPT_B
    coff = jnp.full((16,), c * N, jnp.int32)
    nch = _EPT_B // _CHUNK_B

    def fire_idx(cur, b):
        off = ebase + cur * _CHUNK_B
        pltpu.async_copy(ei_hbm.at[:, pl.ds(off, _CHUNK_B)], eib[b], semi[b])
        pltpu.async_copy(scores_hbm.at[pl.ds(off, _CHUNK_B)], scb[b], semi[b])

    def wait_idx(b):
        pltpu.make_async_copy(ei_hbm.at[:, pl.ds(0, _CHUNK_B)], eib[b],
                              semi[b]).wait()
        pltpu.make_async_copy(scores_hbm.at[pl.ds(0, _CHUNK_B)], scb[b],
                              semi[b]).wait()

    def vec_compute(b):
        def vec(g, _):
            sl = pl.ds(g * 16, 16)
            evb[b][sl] = jnp.exp(scb[b][sl] - gmax)
            ixb[b][sl] = eib[b][0, sl] + coff
            dstb[b][sl] = eib[b][1, sl]
            return 0

        lax.fori_loop(0, _CHUNK_B // 16, vec, 0)

    def fire_gather(b):
        pltpu.async_copy(vflat_hbm.at[ixb[b]], vrb[b], semg[b])

    def wait_gather(b):
        pltpu.make_async_copy(vflat_hbm.at[pl.ds(0, _CHUNK_B)], vrb[b],
                              semg[b]).wait()

    def wait_scatter_pair():
        pltpu.make_async_copy(vr0, agg_sh.at[pl.ds(0, _CHUNK_B)], sems).wait()
        pltpu.make_async_copy(ev0, den_sh.at[pl.ds(0, _CHUNK_B)], sems).wait()

    def process(b):
        wait_gather(b)

        def edge(e, _):
            eb = plsc.load_gather(evb[b], [jnp.full((16,), e, jnp.int32)])
            for j in range(_HH // 16):
                sl = pl.ds(j * 16, 16)
                vrb[b][e, sl] = vrb[b][e, sl] * eb
            return 0

        lax.fori_loop(0, _CHUNK_B, edge, 0)
        pltpu.async_copy(vrb[b], agg_sh.at[dstb[b]], sems, add=True)
        pltpu.async_copy(evb[b], den_sh.at[dstb[b]], sems, add=True)

    fire_idx(0, 0)
    fire_idx(1, 1)
    wait_idx(0)
    vec_compute(0)
    fire_idx(2, 0)
    fire_gather(0)

    def body(k2, _):
        for b2 in (0, 1):
            ci = 2 * k2 + b2

            @pl.when(ci >= 1)
            def _():
                wait_scatter_pair()

            wait_idx(1 - b2)
            vec_compute(1 - b2)

            @pl.when(ci + 3 < nch)
            def _():
                fire_idx(ci + 3, 1 - b2)

            fire_gather(1 - b2)
            process(b2)
        return 0

    lax.fori_loop(0, (nch - 2) // 2, body, 0)
    wait_scatter_pair()
    wait_idx(1)
    vec_compute(1)
    fire_gather(1)
    process(0)
    wait_scatter_pair()
    process(1)
    wait_scatter_pair()
    plsc.subcore_barrier()

    rbase = t * _RPT

    def nchunk(rc, _):
        r0 = rbase + rc * _RCH
        pltpu.sync_copy(agg_sh.at[pl.ds(r0, _RCH)], arows)
        pltpu.sync_copy(den_sh.at[pl.ds(r0, _RCH)], den_v)

        def row(r, _):
            db = plsc.load_gather(den_v, [jnp.full((16,), r, jnp.int32)]) + 1e-16
            for j in range(_HH // 16):
                sl = pl.ds(j * 16, 16)
                arows[r, sl] = arows[r, sl] / db
            return 0

        lax.fori_loop(0, _RCH, row, 0)
        pltpu.sync_copy(arows, out_hbm.at[pl.ds(c * NP + r0, _RCH)])
        return 0

    lax.fori_loop(0, _RPT // _RCH, nchunk, 0)


def _sc_aggregate(scores, pmax, edge_index, vflat):
    mesh = plsc.VectorSubcoreMesh(core_axis_name="c", subcore_axis_name="s",
                                  num_cores=NC, num_subcores=NS)
    kern = pl.kernel(
        _agg_body,
        out_type=jax.ShapeDtypeStruct((2 * NP, _HH), jnp.float32),
        mesh=mesh,
        compiler_params=pltpu.CompilerParams(use_tc_tiling_on_sc=False, needs_layout_passes=False),
        scratch_types=[
            pltpu.VMEM((2, _CHUNK_B), jnp.int32),
            pltpu.VMEM((2, _CHUNK_B), jnp.int32),
            pltpu.VMEM((_CHUNK_B,), jnp.int32),
            pltpu.VMEM((_CHUNK_B,), jnp.int32),
            pltpu.VMEM((_CHUNK_B,), jnp.int32),
            pltpu.VMEM((_CHUNK_B,), jnp.int32),
            pltpu.VMEM((_CHUNK_B,), jnp.float32),
            pltpu.VMEM((_CHUNK_B,), jnp.float32),
            pltpu.VMEM((_CHUNK_B, _HH), jnp.float32),
            pltpu.VMEM((_CHUNK_B, _HH), jnp.float32),
            pltpu.VMEM((_CHUNK_B,), jnp.float32),
            pltpu.VMEM((_CHUNK_B,), jnp.float32),
            pltpu.VMEM((_RCH, _HH), jnp.float32),
            pltpu.VMEM((_RCH,), jnp.float32),
            pltpu.VMEM((NT * 16,), jnp.float32),
            pltpu.VMEM_SHARED((NP, _HH), jnp.float32),
            pltpu.VMEM_SHARED((NP,), jnp.float32),
            pltpu.SemaphoreType.DMA,
            pltpu.SemaphoreType.DMA,
            pltpu.SemaphoreType.DMA,
            pltpu.SemaphoreType.DMA,
            pltpu.SemaphoreType.DMA,
        ],
    )
    return kern(scores, pmax, edge_index, vflat)


def kernel(x, edge_index, W_emb, b_emb, Wq, bq, Wk, bk, Wv, bv, Ws, bs):
    h = _matmul_bias(x, W_emb, b_emb)

    wcat = jnp.concatenate([Wq[1:], Wk[1:], Wv[1:], Ws[1:]], axis=1)
    ts = np.linspace(0.0, 1.0, N_STEPS).astype(np.float32)

    ys = [h]
    y = h
    for i in range(N_STEPS - 1):
        tcur = float(ts[i])
        dt = float(ts[i + 1] - ts[i])
        bcat = jnp.concatenate(
            [bq + tcur * Wq[0], bk + tcur * Wk[0],
             bv + tcur * Wv[0], bs + tcur * Ws[0]]
        )
        q, k, v0, v1, s = _qkvs(y, wcat, bcat)
        vflat = jnp.concatenate([v0, v1], axis=0)
        scores, pmax = _sc_scores(q, k, edge_index)
        aggflat = _sc_aggregate(scores, pmax, edge_index, vflat)
        aggn = jnp.concatenate([aggflat[:N], aggflat[NP:NP + N]], axis=1)
        y = _euler_update(y, aggn, s, dt)
        ys.append(y)
    return jnp.stack(ys, axis=0)


# speedup vs baseline: 11.0217x; 1.0013x over previous
import jax
import jax.numpy as jnp
import numpy as np
from jax import lax
from jax.experimental import pallas as pl
from jax.experimental.pallas import tpu as pltpu
from jax.experimental.pallas import tpu_sc as plsc

N = 10000
E = 320000
D_IN = 128
H = 256
N_STEPS = 4
NP = 10240
NC, NS = 2, 16
NT = NC * NS
SCALE = 0.0625


def _matmul_bias(xa, w, b, bm=2000):
    n, kd = xa.shape
    m = w.shape[1]

    def body(x_ref, w_ref, b_ref, o_ref):
        o_ref[...] = (
            jnp.dot(x_ref[...], w_ref[...], preferred_element_type=jnp.float32)
            + b_ref[...]
        )

    return pl.pallas_call(
        body,
        grid=(n // bm,),
        in_specs=[
            pl.BlockSpec((bm, kd), lambda i: (i, 0)),
            pl.BlockSpec((kd, m), lambda i: (0, 0)),
            pl.BlockSpec((1, m), lambda i: (0, 0)),
        ],
        out_specs=pl.BlockSpec((bm, m), lambda i: (i, 0)),
        out_shape=jax.ShapeDtypeStruct((n, m), jnp.float32),
    )(xa, w, b.reshape(1, m))


_QKVS_OUT_SPECS = [
    pl.BlockSpec((2000, H), lambda i: (i, 0)),
    pl.BlockSpec((2000, H), lambda i: (i, 0)),
    pl.BlockSpec((2000, H // 2), lambda i: (i, 0)),
    pl.BlockSpec((2000, H // 2), lambda i: (i, 0)),
    pl.BlockSpec((2000, H), lambda i: (i, 0)),
]
_QKVS_OUT_SHAPES = [
    jax.ShapeDtypeStruct((N, H), jnp.float32),
    jax.ShapeDtypeStruct((N, H), jnp.float32),
    jax.ShapeDtypeStruct((N, H // 2), jnp.float32),
    jax.ShapeDtypeStruct((N, H // 2), jnp.float32),
    jax.ShapeDtypeStruct((N, H), jnp.float32),
]


def _write_qkvs(acc, q_ref, k_ref, v0_ref, v1_ref, s_ref):
    q_ref[...] = acc[:, 0:256]
    k_ref[...] = acc[:, 256:512]
    v0_ref[...] = acc[:, 512:640]
    v1_ref[...] = acc[:, 640:768]
    s_ref[...] = acc[:, 768:1024]


def _embed_qkvs(x, W_emb, b_emb, wcat, bcat, bm=2000):
    def body(x_ref, we_ref, be_ref, w_ref, b_ref, h_ref,
             q_ref, k_ref, v0_ref, v1_ref, s_ref):
        hblk = (
            jnp.dot(x_ref[...], we_ref[...], preferred_element_type=jnp.float32)
            + be_ref[...]
        )
        h_ref[...] = hblk
        acc = (
            jnp.dot(hblk, w_ref[...], preferred_element_type=jnp.float32)
            + b_ref[...]
        )
        _write_qkvs(acc, q_ref, k_ref, v0_ref, v1_ref, s_ref)

    return pl.pallas_call(
        body,
        grid=(N // bm,),
        in_specs=[
            pl.BlockSpec((bm, D_IN), lambda i: (i, 0)),
            pl.BlockSpec((D_IN, H), lambda i: (0, 0)),
            pl.BlockSpec((1, H), lambda i: (0, 0)),
            pl.BlockSpec((H, 4 * H), lambda i: (0, 0)),
            pl.BlockSpec((1, 4 * H), lambda i: (0, 0)),
        ],
        out_specs=[pl.BlockSpec((bm, H), lambda i: (i, 0))] + _QKVS_OUT_SPECS,
        out_shape=[jax.ShapeDtypeStruct((N, H), jnp.float32)] + _QKVS_OUT_SHAPES,
    )(x, W_emb, b_emb.reshape(1, H), wcat, bcat.reshape(1, 4 * H))


def _update_qkvs(y, aggf, s, dt, wcat, bcat, bm=2000):
    def body(y_ref, a_ref, s_ref, w_ref, b_ref, yn_ref,
             q_ref, k_ref, v0_ref, v1_ref, sn_ref):
        ynew = y_ref[...] + dt * (a_ref[...] + s_ref[...])
        yn_ref[...] = ynew
        acc = (
            jnp.dot(ynew, w_ref[...], preferred_element_type=jnp.float32)
            + b_ref[...]
        )
        _write_qkvs(acc, q_ref, k_ref, v0_ref, v1_ref, sn_ref)

    return pl.pallas_call(
        body,
        grid=(N // bm,),
        in_specs=[
            pl.BlockSpec((bm, H), lambda i: (i, 0)),
            pl.BlockSpec((bm, H), lambda i: (i, 0)),
            pl.BlockSpec((bm, H), lambda i: (i, 0)),
            pl.BlockSpec((H, 4 * H), lambda i: (0, 0)),
            pl.BlockSpec((1, 4 * H), lambda i: (0, 0)),
        ],
        out_specs=[pl.BlockSpec((bm, H), lambda i: (i, 0))] + _QKVS_OUT_SPECS,
        out_shape=[jax.ShapeDtypeStruct((N, H), jnp.float32)] + _QKVS_OUT_SHAPES,
    )(y, aggf, s, wcat, bcat.reshape(1, 4 * H))


def _euler_update(y, aggf, s, dt, bm=2000):
    def body(y_ref, a_ref, s_ref, o_ref):
        o_ref[...] = y_ref[...] + dt * (a_ref[...] + s_ref[...])

    return pl.pallas_call(
        body,
        grid=(N // bm,),
        in_specs=[pl.BlockSpec((bm, H), lambda i: (i, 0))] * 3,
        out_specs=pl.BlockSpec((bm, H), lambda i: (i, 0)),
        out_shape=jax.ShapeDtypeStruct((N, H), jnp.float32),
    )(y, aggf, s)


_CHUNK_A = 80
_EPT_A = E // NT
_CHUNK_B = 80
_EPT_B = E // NS
_RPT = NP // NS
_RCH = 64
_HH = H // 2


def _score_body(q_hbm, k_hbm, ei_hbm, scores_hbm, pmax_hbm,
                ei0, ei1, qr0, qr1, kr0, kr1, sv0, sv1, mx_v,
                sem0, sem1, semi0, semi1, semw):
    wid = lax.axis_index("s") * NC + lax.axis_index("c")
    base = wid * _EPT_A
    lanes = lax.iota(jnp.int32, 16)
    ones = jnp.full((16,), 1, jnp.int32)
    eib = (ei0, ei1)
    qrb = (qr0, qr1)
    krb = (kr0, kr1)
    svb = (sv0, sv1)
    semb = (sem0, sem1)
    semi = (semi0, semi1)
    nch = _EPT_A // _CHUNK_A

    def fire_idx(cur, b):
        off = base + cur * _CHUNK_A
        pltpu.async_copy(ei_hbm.at[:, pl.ds(off, _CHUNK_A)], eib[b], semi[b])

    def wait_idx(b):
        pltpu.make_async_copy(ei_hbm.at[:, pl.ds(0, _CHUNK_A)], eib[b],
                              semi[b]).wait()

    def fire_gathers(b):
        pltpu.async_copy(q_hbm.at[eib[b].at[1]], qrb[b], semb[b])
        pltpu.async_copy(k_hbm.at[eib[b].at[0]], krb[b], semb[b])

    def wait_gathers(b):
        pltpu.make_async_copy(q_hbm.at[pl.ds(0, _CHUNK_A)], qrb[b], semb[b]).wait()
        pltpu.make_async_copy(k_hbm.at[pl.ds(0, _CHUNK_A)], krb[b], semb[b]).wait()

    def compute(cur, b, mv):
        off = base + cur * _CHUNK_A

        @pl.when(cur >= 2)
        def _():
            pltpu.make_async_copy(svb[b], scores_hbm.at[pl.ds(0, _CHUNK_A)],
                                  semw).wait()

        def grp(g, mcur):
            eidx = lanes + g * 16
            zf = jnp.zeros((16,), jnp.float32)

            def jblk(jb, carry):
                cidx, a0, a1, a2, a3 = carry
                accs = [a0, a1, a2, a3]
                for u in range(16):
                    qv = plsc.load_gather(qrb[b], [eidx, cidx])
                    kv = plsc.load_gather(krb[b], [eidx, cidx])
                    accs[u % 4] = accs[u % 4] + qv * kv
                    cidx = cidx + ones
                return (cidx, accs[0], accs[1], accs[2], accs[3])

            cidx, a0, a1, a2, a3 = lax.fori_loop(
                0, 15, jblk, (lanes, zf, zf, zf, zf))
            accs = [a0, a1, a2, a3]
            cmod = jnp.full((16,), H, jnp.int32)
            for u in range(16):
                cw = jnp.where(cidx >= cmod, cidx - cmod, cidx)
                qv = plsc.load_gather(qrb[b], [eidx, cw])
                kv = plsc.load_gather(krb[b], [eidx, cw])
                accs[u % 4] = accs[u % 4] + qv * kv
                cidx = cidx + ones
            sc = ((accs[0] + accs[1]) + (accs[2] + accs[3])) * SCALE
            svb[b][pl.ds(g * 16, 16)] = sc
            return jnp.maximum(mcur, sc)

        mv = lax.fori_loop(0, _CHUNK_A // 16, grp, mv)
        pltpu.async_copy(svb[b], scores_hbm.at[pl.ds(off, _CHUNK_A)], semw)
        return mv

    fire_idx(0, 0)
    fire_idx(1, 1)
    wait_idx(0)
    fire_gathers(0)

    def body(k2, mv):
        for b in (0, 1):
            cur = 2 * k2 + b
            wait_gathers(b)

            @pl.when(cur + 2 < nch)
            def _():
                fire_idx(cur + 2, b)

            wait_idx(1 - b)
            fire_gathers(1 - b)
            mv = compute(cur, b, mv)
        return mv

    mv = lax.fori_loop(0, (nch - 1) // 2, body,
                       jnp.full((16,), -3.0e38, jnp.float32))
    wait_gathers(0)
    mv = compute(nch - 1, 0, mv)
    pltpu.make_async_copy(sv0, scores_hbm.at[pl.ds(0, _CHUNK_A)], semw).wait()
    pltpu.make_async_copy(sv1, scores_hbm.at[pl.ds(0, _CHUNK_A)], semw).wait()
    mx_v[...] = mv
    pltpu.sync_copy(mx_v, pmax_hbm.at[pl.ds(wid * 16, 16)])


def _sc_scores(q, k, edge_index):
    mesh = plsc.VectorSubcoreMesh(core_axis_name="c", subcore_axis_name="s",
                                  num_cores=NC, num_subcores=NS)
    kern = pl.kernel(
        _score_body,
        out_type=[
            jax.ShapeDtypeStruct((E,), jnp.float32),
            jax.ShapeDtypeStruct((NT * 16,), jnp.float32),
        ],
        mesh=mesh,
        compiler_params=pltpu.CompilerParams(use_tc_tiling_on_sc=False, needs_layout_passes=False),
        scratch_types=[
            pltpu.VMEM((2, _CHUNK_A), jnp.int32),
            pltpu.VMEM((2, _CHUNK_A), jnp.int32),
            pltpu.VMEM((_CHUNK_A, H), jnp.float32),
            pltpu.VMEM((_CHUNK_A, H), jnp.float32),
            pltpu.VMEM((_CHUNK_A, H), jnp.float32),
            pltpu.VMEM((_CHUNK_A, H), jnp.float32),
            pltpu.VMEM((_CHUNK_A,), jnp.float32),
            pltpu.VMEM((_CHUNK_A,), jnp.float32),
            pltpu.VMEM((16,), jnp.float32),
            pltpu.SemaphoreType.DMA,
            pltpu.SemaphoreType.DMA,
            pltpu.SemaphoreType.DMA,
            pltpu.SemaphoreType.DMA,
            pltpu.SemaphoreType.DMA,
        ],
    )
    return kern(q, k, edge_index)


def _agg_body(scores_hbm, pmax_hbm, ei_hbm, vflat_hbm, out_hbm,
              ei0, ei1, dst0, dst1, ix0, ix1, ev0, ev1, vr0, vr1,
              sc0, sc1, arows, den_v, pm_v, agg_sh, den_sh,
              semg0, semg1, semi0, semi1, sems):
    c = lax.axis_index("c")
    t = lax.axis_index("s")
    eib = (ei0, ei1)
    dstb = (dst0, dst1)
    ixb = (ix0, ix1)
    evb = (ev0, ev1)
    vrb = (vr0, vr1)
    scb = (sc0, sc1)
    semg = (semg0, semg1)
    semi = (semi0, semi1)

    def zrow(i, _):
        for j in range(_HH // 16):
            arows[i, pl.ds(j * 16, 16)] = jnp.zeros((16,), jnp.float32)
        return 0

    lax.fori_loop(0, _RCH, zrow, 0)

    def zev(g, _):
        ev0[pl.ds(g * 16, 16)] = jnp.zeros((16,), jnp.float32)
        return 0

    lax.fori_loop(0, _CHUNK_B // 16, zev, 0)

    def zagg(rc, _):
        pltpu.sync_copy(arows, agg_sh.at[pl.ds(t * _RPT + rc * _RCH, _RCH)])
        return 0

    lax.fori_loop(0, _RPT // _RCH, zagg, 0)

    def zden(zi, _):
        pltpu.sync_copy(ev0, den_sh.at[pl.ds(t * _RPT + zi * _CHUNK_B, _CHUNK_B)])
        return 0

    lax.fori_loop(0, _RPT // _CHUNK_B, zden, 0)
    plsc.subcore_barrier()

    pltpu.sync_copy(pmax_hbm, pm_v)
    mv = pm_v[pl.ds(0, 16)]
    for g in range(1, NT):
        mv = jnp.maximum(mv, pm_v[pl.ds(g * 16, 16)])
    gmax = jnp.max(mv)

    ebase = t * _EPT_B
    coff = jnp.full((16,), c * N, jnp.int32)
    nch = _EPT_B // _CHUNK_B

    def fire_idx(cur, b):
        off = ebase + cur * _CHUNK_B
        pltpu.async_copy(ei_hbm.at[:, pl.ds(off, _CHUNK_B)], eib[b], semi[b])
        pltpu.async_copy(scores_hbm.at[pl.ds(off, _CHUNK_B)], scb[b], semi[b])

    def wait_idx(b):
        pltpu.make_async_copy(ei_hbm.at[:, pl.ds(0, _CHUNK_B)], eib[b],
                              semi[b]).wait()
        pltpu.make_async_copy(scores_hbm.at[pl.ds(0, _CHUNK_B)], scb[b],
                              semi[b]).wait()

    def vec_compute(b):
        def vec(g, _):
            sl = pl.ds(g * 16, 16)
            evb[b][sl] = jnp.exp(scb[b][sl] - gmax)
            ixb[b][sl] = eib[b][0, sl] + coff
            dstb[b][sl] = eib[b][1, sl]
            return 0

        lax.fori_loop(0, _CHUNK_B // 16, vec, 0)

    def fire_gather(b):
        pltpu.async_copy(vflat_hbm.at[ixb[b]], vrb[b], semg[b])

    def wait_gather(b):
        pltpu.make_async_copy(vflat_hbm.at[pl.ds(0, _CHUNK_B)], vrb[b],
                              semg[b]).wait()

    def wait_scatter_pair():
        pltpu.make_async_copy(vr0, agg_sh.at[pl.ds(0, _CHUNK_B)], sems).wait()
        pltpu.make_async_copy(ev0, den_sh.at[pl.ds(0, _CHUNK_B)], sems).wait()

    def process(b):
        wait_gather(b)

        def edge(e, _):
            eb = plsc.load_gather(evb[b], [jnp.full((16,), e, jnp.int32)])
            for j in range(_HH // 16):
                sl = pl.ds(j * 16, 16)
                vrb[b][e, sl] = vrb[b][e, sl] * eb
            return 0

        lax.fori_loop(0, _CHUNK_B, edge, 0)
        pltpu.async_copy(vrb[b], agg_sh.at[dstb[b]], sems, add=True)
        pltpu.async_copy(evb[b], den_sh.at[dstb[b]], sems, add=True)

    fire_idx(0, 0)
    fire_idx(1, 1)
    wait_idx(0)
    vec_compute(0)
    fire_idx(2, 0)
    fire_gather(0)

    def body(k2, _):
        for b2 in (0, 1):
            ci = 2 * k2 + b2

            @pl.when(ci >= 1)
            def _():
                wait_scatter_pair()

            wait_idx(1 - b2)
            vec_compute(1 - b2)

            @pl.when(ci + 3 < nch)
            def _():
                fire_idx(ci + 3, 1 - b2)

            fire_gather(1 - b2)
            process(b2)
        return 0

    lax.fori_loop(0, (nch - 2) // 2, body, 0)
    wait_scatter_pair()
    wait_idx(1)
    vec_compute(1)
    fire_gather(1)
    process(0)
    wait_scatter_pair()
    process(1)
    wait_scatter_pair()
    plsc.subcore_barrier()

    rbase = t * _RPT

    def nchunk(rc, _):
        r0 = rbase + rc * _RCH
        pltpu.sync_copy(agg_sh.at[pl.ds(r0, _RCH)], arows)
        pltpu.sync_copy(den_sh.at[pl.ds(r0, _RCH)], den_v)

        def row(r, _):
            db = plsc.load_gather(den_v, [jnp.full((16,), r, jnp.int32)]) + 1e-16
            for j in range(_HH // 16):
                sl = pl.ds(j * 16, 16)
                arows[r, sl] = arows[r, sl] / db
            return 0

        lax.fori_loop(0, _RCH, row, 0)
        pltpu.sync_copy(arows, out_hbm.at[pl.ds(r0, _RCH), pl.ds(c * _HH, _HH)])
        return 0

    lax.fori_loop(0, _RPT // _RCH, nchunk, 0)


def _sc_aggregate(scores, pmax, edge_index, vflat):
    mesh = plsc.VectorSubcoreMesh(core_axis_name="c", subcore_axis_name="s",
                                  num_cores=NC, num_subcores=NS)
    kern = pl.kernel(
        _agg_body,
        out_type=jax.ShapeDtypeStruct((NP, H), jnp.float32),
        mesh=mesh,
        compiler_params=pltpu.CompilerParams(use_tc_tiling_on_sc=False, needs_layout_passes=False),
        scratch_types=[
            pltpu.VMEM((2, _CHUNK_B), jnp.int32),
            pltpu.VMEM((2, _CHUNK_B), jnp.int32),
            pltpu.VMEM((_CHUNK_B,), jnp.int32),
            pltpu.VMEM((_CHUNK_B,), jnp.int32),
            pltpu.VMEM((_CHUNK_B,), jnp.int32),
            pltpu.VMEM((_CHUNK_B,), jnp.int32),
            pltpu.VMEM((_CHUNK_B,), jnp.float32),
            pltpu.VMEM((_CHUNK_B,), jnp.float32),
            pltpu.VMEM((_CHUNK_B, _HH), jnp.float32),
            pltpu.VMEM((_CHUNK_B, _HH), jnp.float32),
            pltpu.VMEM((_CHUNK_B,), jnp.float32),
            pltpu.VMEM((_CHUNK_B,), jnp.float32),
            pltpu.VMEM((_RCH, _HH), jnp.float32),
            pltpu.VMEM((_RCH,), jnp.float32),
            pltpu.VMEM((NT * 16,), jnp.float32),
            pltpu.VMEM_SHARED((NP, _HH), jnp.float32),
            pltpu.VMEM_SHARED((NP,), jnp.float32),
            pltpu.SemaphoreType.DMA,
            pltpu.SemaphoreType.DMA,
            pltpu.SemaphoreType.DMA,
            pltpu.SemaphoreType.DMA,
            pltpu.SemaphoreType.DMA,
        ],
    )
    return kern(scores, pmax, edge_index, vflat)


def kernel(x, edge_index, W_emb, b_emb, Wq, bq, Wk, bk, Wv, bv, Ws, bs):
    wcat = jnp.concatenate([Wq[1:], Wk[1:], Wv[1:], Ws[1:]], axis=1)
    ts = np.linspace(0.0, 1.0, N_STEPS).astype(np.float32)
    bcats = [
        jnp.concatenate(
            [bq + float(t) * Wq[0], bk + float(t) * Wk[0],
             bv + float(t) * Wv[0], bs + float(t) * Ws[0]]
        )
        for t in ts[:N_STEPS - 1]
    ]

    h, q, k, v0, v1, s = _embed_qkvs(x, W_emb, b_emb, wcat, bcats[0])
    ys = [h]
    y = h
    for i in range(N_STEPS - 1):
        dt = float(ts[i + 1] - ts[i])
        vflat = jnp.concatenate([v0, v1], axis=0)
        scores, pmax = _sc_scores(q, k, edge_index)
        aggf = _sc_aggregate(scores, pmax, edge_index, vflat)
        if i < N_STEPS - 2:
            y, q, k, v0, v1, s = _update_qkvs(y, aggf, s, dt, wcat, bcats[i + 1])
        else:
            y = _euler_update(y, aggf, s, dt)
        ys.append(y)
    return jnp.stack(ys, axis=0)


# speedup vs baseline: 11.8243x; 1.0728x over previous
import jax
import jax.numpy as jnp
import numpy as np
from jax import lax
from jax.experimental import pallas as pl
from jax.experimental.pallas import tpu as pltpu
from jax.experimental.pallas import tpu_sc as plsc

N = 10000
E = 320000
D_IN = 128
H = 256
N_STEPS = 4
NP = 10240
NC, NS = 2, 16
NT = NC * NS
SCALE = 0.0625


def _matmul_bias(xa, w, b, bm=2000):
    n, kd = xa.shape
    m = w.shape[1]

    def body(x_ref, w_ref, b_ref, o_ref):
        o_ref[...] = (
            jnp.dot(x_ref[...], w_ref[...], preferred_element_type=jnp.float32)
            + b_ref[...]
        )

    return pl.pallas_call(
        body,
        grid=(n // bm,),
        in_specs=[
            pl.BlockSpec((bm, kd), lambda i: (i, 0)),
            pl.BlockSpec((kd, m), lambda i: (0, 0)),
            pl.BlockSpec((1, m), lambda i: (0, 0)),
        ],
        out_specs=pl.BlockSpec((bm, m), lambda i: (i, 0)),
        out_shape=jax.ShapeDtypeStruct((n, m), jnp.float32),
    )(xa, w, b.reshape(1, m))


_QKVS_OUT_SPECS = [
    pl.BlockSpec((2000, H), lambda i: (i, 0)),
    pl.BlockSpec((2000, H), lambda i: (i, 0)),
    pl.BlockSpec((2000, H // 2), lambda i: (i, 0)),
    pl.BlockSpec((2000, H // 2), lambda i: (i, 0)),
    pl.BlockSpec((2000, H), lambda i: (i, 0)),
]
_QKVS_OUT_SHAPES = [
    jax.ShapeDtypeStruct((N, H), jnp.float32),
    jax.ShapeDtypeStruct((N, H), jnp.float32),
    jax.ShapeDtypeStruct((N, H // 2), jnp.float32),
    jax.ShapeDtypeStruct((N, H // 2), jnp.float32),
    jax.ShapeDtypeStruct((N, H), jnp.float32),
]


def _write_qkvs(acc, q_ref, k_ref, v0_ref, v1_ref, s_ref):
    q_ref[...] = acc[:, 0:256]
    k_ref[...] = acc[:, 256:512]
    v0_ref[...] = acc[:, 512:640]
    v1_ref[...] = acc[:, 640:768]
    s_ref[...] = acc[:, 768:1024]


def _embed_qkvs(x, W_emb, b_emb, wcat, bcat, bm=2000):
    def body(x_ref, we_ref, be_ref, w_ref, b_ref, h_ref,
             q_ref, k_ref, v0_ref, v1_ref, s_ref):
        hblk = (
            jnp.dot(x_ref[...], we_ref[...], preferred_element_type=jnp.float32)
            + be_ref[...]
        )
        h_ref[...] = hblk
        acc = (
            jnp.dot(hblk, w_ref[...], preferred_element_type=jnp.float32)
            + b_ref[...]
        )
        _write_qkvs(acc, q_ref, k_ref, v0_ref, v1_ref, s_ref)

    return pl.pallas_call(
        body,
        grid=(N // bm,),
        in_specs=[
            pl.BlockSpec((bm, D_IN), lambda i: (i, 0)),
            pl.BlockSpec((D_IN, H), lambda i: (0, 0)),
            pl.BlockSpec((1, H), lambda i: (0, 0)),
            pl.BlockSpec((H, 4 * H), lambda i: (0, 0)),
            pl.BlockSpec((1, 4 * H), lambda i: (0, 0)),
        ],
        out_specs=[pl.BlockSpec((bm, H), lambda i: (i, 0))] + _QKVS_OUT_SPECS,
        out_shape=[jax.ShapeDtypeStruct((N, H), jnp.float32)] + _QKVS_OUT_SHAPES,
    )(x, W_emb, b_emb.reshape(1, H), wcat, bcat.reshape(1, 4 * H))


def _update_qkvs(y, aggf, s, dt, wcat, bcat, bm=2000):
    def body(y_ref, a_ref, s_ref, w_ref, b_ref, yn_ref,
             q_ref, k_ref, v0_ref, v1_ref, sn_ref):
        ynew = y_ref[...] + dt * (a_ref[...] + s_ref[...])
        yn_ref[...] = ynew
        acc = (
            jnp.dot(ynew, w_ref[...], preferred_element_type=jnp.float32)
            + b_ref[...]
        )
        _write_qkvs(acc, q_ref, k_ref, v0_ref, v1_ref, sn_ref)

    return pl.pallas_call(
        body,
        grid=(N // bm,),
        in_specs=[
            pl.BlockSpec((bm, H), lambda i: (i, 0)),
            pl.BlockSpec((bm, H), lambda i: (i, 0)),
            pl.BlockSpec((bm, H), lambda i: (i, 0)),
            pl.BlockSpec((H, 4 * H), lambda i: (0, 0)),
            pl.BlockSpec((1, 4 * H), lambda i: (0, 0)),
        ],
        out_specs=[pl.BlockSpec((bm, H), lambda i: (i, 0))] + _QKVS_OUT_SPECS,
        out_shape=[jax.ShapeDtypeStruct((N, H), jnp.float32)] + _QKVS_OUT_SHAPES,
    )(y, aggf, s, wcat, bcat.reshape(1, 4 * H))


def _euler_update(y, aggf, s, dt, bm=2000):
    def body(y_ref, a_ref, s_ref, o_ref):
        o_ref[...] = y_ref[...] + dt * (a_ref[...] + s_ref[...])

    return pl.pallas_call(
        body,
        grid=(N // bm,),
        in_specs=[pl.BlockSpec((bm, H), lambda i: (i, 0))] * 3,
        out_specs=pl.BlockSpec((bm, H), lambda i: (i, 0)),
        out_shape=jax.ShapeDtypeStruct((N, H), jnp.float32),
    )(y, aggf, s)


_CHUNK_A = 80
_EPT_A = E // NT
_CHUNK_B = 80
_EPT_B = E // NS
_RPT = NP // NS
_RCH = 64
_HH = H // 2


def _score_body(q_hbm, k_hbm, ei_hbm, scores_hbm, pmax_hbm,
                ei0, ei1, qr0, qr1, kr0, kr1, sv0, sv1, mx_v,
                sem0, sem1, semi0, semi1, semw):
    wid = lax.axis_index("s") * NC + lax.axis_index("c")
    base = wid * _EPT_A
    lanes = lax.iota(jnp.int32, 16)
    ones = jnp.full((16,), 1, jnp.int32)
    eib = (ei0, ei1)
    qrb = (qr0, qr1)
    krb = (kr0, kr1)
    svb = (sv0, sv1)
    semb = (sem0, sem1)
    semi = (semi0, semi1)
    nch = _EPT_A // _CHUNK_A

    def fire_idx(cur, b):
        off = base + cur * _CHUNK_A
        pltpu.async_copy(ei_hbm.at[:, pl.ds(off, _CHUNK_A)], eib[b], semi[b])

    def wait_idx(b):
        pltpu.make_async_copy(ei_hbm.at[:, pl.ds(0, _CHUNK_A)], eib[b],
                              semi[b]).wait()

    def fire_gathers(b):
        pltpu.async_copy(q_hbm.at[eib[b].at[1]], qrb[b], semb[b])
        pltpu.async_copy(k_hbm.at[eib[b].at[0]], krb[b], semb[b])

    def wait_gathers(b):
        pltpu.make_async_copy(q_hbm.at[pl.ds(0, _CHUNK_A)], qrb[b], semb[b]).wait()
        pltpu.make_async_copy(k_hbm.at[pl.ds(0, _CHUNK_A)], krb[b], semb[b]).wait()

    def compute(cur, b, mv):
        off = base + cur * _CHUNK_A

        @pl.when(cur >= 2)
        def _():
            pltpu.make_async_copy(svb[b], scores_hbm.at[pl.ds(0, _CHUNK_A)],
                                  semw).wait()

        def grp(g, mcur):
            eidx = lanes + g * 16
            zf = jnp.zeros((16,), jnp.float32)

            def jblk(jb, carry):
                cidx = carry[0]
                accs = list(carry[1:])
                for u in range(16):
                    qv = plsc.load_gather(qrb[b], [eidx, cidx])
                    kv = plsc.load_gather(krb[b], [eidx, cidx])
                    accs[u % 8] = accs[u % 8] + qv * kv
                    cidx = cidx + ones
                return (cidx, *accs)

            carry = lax.fori_loop(
                0, 15, jblk, (lanes,) + (zf,) * 8)
            cidx = carry[0]
            accs = list(carry[1:])
            cmod = jnp.full((16,), H, jnp.int32)
            for u in range(16):
                cw = jnp.where(cidx >= cmod, cidx - cmod, cidx)
                qv = plsc.load_gather(qrb[b], [eidx, cw])
                kv = plsc.load_gather(krb[b], [eidx, cw])
                accs[u % 8] = accs[u % 8] + qv * kv
                cidx = cidx + ones
            sc = (((accs[0] + accs[1]) + (accs[2] + accs[3]))
                  + ((accs[4] + accs[5]) + (accs[6] + accs[7]))) * SCALE
            svb[b][pl.ds(g * 16, 16)] = sc
            return jnp.maximum(mcur, sc)

        mv = lax.fori_loop(0, _CHUNK_A // 16, grp, mv)
        pltpu.async_copy(svb[b], scores_hbm.at[pl.ds(off, _CHUNK_A)], semw)
        return mv

    fire_idx(0, 0)
    fire_idx(1, 1)
    wait_idx(0)
    fire_gathers(0)

    def body(k2, mv):
        for b in (0, 1):
            cur = 2 * k2 + b
            wait_gathers(b)

            @pl.when(cur + 2 < nch)
            def _():
                fire_idx(cur + 2, b)

            wait_idx(1 - b)
            fire_gathers(1 - b)
            mv = compute(cur, b, mv)
        return mv

    mv = lax.fori_loop(0, (nch - 1) // 2, body,
                       jnp.full((16,), -3.0e38, jnp.float32))
    wait_gathers(0)
    mv = compute(nch - 1, 0, mv)
    pltpu.make_async_copy(sv0, scores_hbm.at[pl.ds(0, _CHUNK_A)], semw).wait()
    pltpu.make_async_copy(sv1, scores_hbm.at[pl.ds(0, _CHUNK_A)], semw).wait()
    mx_v[...] = mv
    pltpu.sync_copy(mx_v, pmax_hbm.at[pl.ds(wid * 16, 16)])


def _sc_scores(q, k, edge_index):
    mesh = plsc.VectorSubcoreMesh(core_axis_name="c", subcore_axis_name="s",
                                  num_cores=NC, num_subcores=NS)
    kern = pl.kernel(
        _score_body,
        out_type=[
            jax.ShapeDtypeStruct((E,), jnp.float32),
            jax.ShapeDtypeStruct((NT * 16,), jnp.float32),
        ],
        mesh=mesh,
        compiler_params=pltpu.CompilerParams(use_tc_tiling_on_sc=False, needs_layout_passes=False),
        scratch_types=[
            pltpu.VMEM((2, _CHUNK_A), jnp.int32),
            pltpu.VMEM((2, _CHUNK_A), jnp.int32),
            pltpu.VMEM((_CHUNK_A, H), jnp.float32),
            pltpu.VMEM((_CHUNK_A, H), jnp.float32),
            pltpu.VMEM((_CHUNK_A, H), jnp.float32),
            pltpu.VMEM((_CHUNK_A, H), jnp.float32),
            pltpu.VMEM((_CHUNK_A,), jnp.float32),
            pltpu.VMEM((_CHUNK_A,), jnp.float32),
            pltpu.VMEM((16,), jnp.float32),
            pltpu.SemaphoreType.DMA,
            pltpu.SemaphoreType.DMA,
            pltpu.SemaphoreType.DMA,
            pltpu.SemaphoreType.DMA,
            pltpu.SemaphoreType.DMA,
        ],
    )
    return kern(q, k, edge_index)


def _agg_body(scores_hbm, pmax_hbm, ei_hbm, vflat_hbm, out_hbm,
              ei0, ei1, dst0, dst1, ix0, ix1, ev0, ev1, vr0, vr1,
              sc0, sc1, arows, den_v, pm_v, agg_sh, den_sh,
              semg0, semg1, semi0, semi1, sems):
    c = lax.axis_index("c")
    t = lax.axis_index("s")
    eib = (ei0, ei1)
    dstb = (dst0, dst1)
    ixb = (ix0, ix1)
    evb = (ev0, ev1)
    vrb = (vr0, vr1)
    scb = (sc0, sc1)
    semg = (semg0, semg1)
    semi = (semi0, semi1)

    def zrow(i, _):
        for j in range(_HH // 16):
            arows[i, pl.ds(j * 16, 16)] = jnp.zeros((16,), jnp.float32)
        return 0

    lax.fori_loop(0, _RCH, zrow, 0)

    def zev(g, _):
        ev0[pl.ds(g * 16, 16)] = jnp.zeros((16,), jnp.float32)
        return 0

    lax.fori_loop(0, _CHUNK_B // 16, zev, 0)

    def zagg(rc, _):
        pltpu.sync_copy(arows, agg_sh.at[pl.ds(t * _RPT + rc * _RCH, _RCH)])
        return 0

    lax.fori_loop(0, _RPT // _RCH, zagg, 0)

    def zden(zi, _):
        pltpu.sync_copy(ev0, den_sh.at[pl.ds(t * _RPT + zi * _CHUNK_B, _CHUNK_B)])
        return 0

    lax.fori_loop(0, _RPT // _CHUNK_B, zden, 0)
    plsc.subcore_barrier()

    pltpu.sync_copy(pmax_hbm, pm_v)
    mv = pm_v[pl.ds(0, 16)]
    for g in range(1, NT):
        mv = jnp.maximum(mv, pm_v[pl.ds(g * 16, 16)])
    gmax = jnp.max(mv)

    ebase = t * _EPT_B
    coff = jnp.full((16,), c * N, jnp.int32)
    nch = _EPT_B // _CHUNK_B

    def fire_idx(cur, b):
        off = ebase + cur * _CHUNK_B
        pltpu.async_copy(ei_hbm.at[:, pl.ds(off, _CHUNK_B)], eib[b], semi[b])
        pltpu.async_copy(scores_hbm.at[pl.ds(off, _CHUNK_B)], scb[b], semi[b])

    def wait_idx(b):
        pltpu.make_async_copy(ei_hbm.at[:, pl.ds(0, _CHUNK_B)], eib[b],
                              semi[b]).wait()
        pltpu.make_async_copy(scores_hbm.at[pl.ds(0, _CHUNK_B)], scb[b],
                              semi[b]).wait()

    def vec_compute(b):
        def vec(g, _):
            sl = pl.ds(g * 16, 16)
            evb[b][sl] = jnp.exp(scb[b][sl] - gmax)
            ixb[b][sl] = eib[b][0, sl] + coff
            dstb[b][sl] = eib[b][1, sl]
            return 0

        lax.fori_loop(0, _CHUNK_B // 16, vec, 0)

    def fire_gather(b):
        pltpu.async_copy(vflat_hbm.at[ixb[b]], vrb[b], semg[b])

    def wait_gather(b):
        pltpu.make_async_copy(vflat_hbm.at[pl.ds(0, _CHUNK_B)], vrb[b],
                              semg[b]).wait()

    def wait_scatter_pair():
        pltpu.make_async_copy(vr0, agg_sh.at[pl.ds(0, _CHUNK_B)], sems).wait()
        pltpu.make_async_copy(ev0, den_sh.at[pl.ds(0, _CHUNK_B)], sems).wait()

    def process(b):
        wait_gather(b)

        def edge(e2, _):
            e = e2 * 2
            eb0 = plsc.load_gather(evb[b], [jnp.full((16,), e, jnp.int32)])
            eb1 = plsc.load_gather(evb[b], [jnp.full((16,), e + 1, jnp.int32)])
            for j in range(_HH // 16):
                sl = pl.ds(j * 16, 16)
                vrb[b][e, sl] = vrb[b][e, sl] * eb0
                vrb[b][e + 1, sl] = vrb[b][e + 1, sl] * eb1
            return 0

        lax.fori_loop(0, _CHUNK_B // 2, edge, 0)
        pltpu.async_copy(vrb[b], agg_sh.at[dstb[b]], sems, add=True)
        pltpu.async_copy(evb[b], den_sh.at[dstb[b]], sems, add=True)

    fire_idx(0, 0)
    fire_idx(1, 1)
    wait_idx(0)
    vec_compute(0)
    fire_idx(2, 0)
    fire_gather(0)

    def body(k2, _):
        for b2 in (0, 1):
            ci = 2 * k2 + b2

            @pl.when(ci >= 1)
            def _():
                wait_scatter_pair()

            wait_idx(1 - b2)
            vec_compute(1 - b2)

            @pl.when(ci + 3 < nch)
            def _():
                fire_idx(ci + 3, 1 - b2)

            fire_gather(1 - b2)
            process(b2)
        return 0

    lax.fori_loop(0, (nch - 2) // 2, body, 0)
    wait_scatter_pair()
    wait_idx(1)
    vec_compute(1)
    fire_gather(1)
    process(0)
    wait_scatter_pair()
    process(1)
    wait_scatter_pair()
    plsc.subcore_barrier()

    rbase = t * _RPT

    def nchunk(rc, _):
        r0 = rbase + rc * _RCH
        pltpu.sync_copy(agg_sh.at[pl.ds(r0, _RCH)], arows)
        pltpu.sync_copy(den_sh.at[pl.ds(r0, _RCH)], den_v)

        def row(r, _):
            db = plsc.load_gather(den_v, [jnp.full((16,), r, jnp.int32)]) + 1e-16
            for j in range(_HH // 16):
                sl = pl.ds(j * 16, 16)
                arows[r, sl] = arows[r, sl] / db
            return 0

        lax.fori_loop(0, _RCH, row, 0)
        pltpu.sync_copy(arows, out_hbm.at[pl.ds(r0, _RCH), pl.ds(c * _HH, _HH)])
        return 0

    lax.fori_loop(0, _RPT // _RCH, nchunk, 0)


def _sc_aggregate(scores, pmax, edge_index, vflat):
    mesh = plsc.VectorSubcoreMesh(core_axis_name="c", subcore_axis_name="s",
                                  num_cores=NC, num_subcores=NS)
    kern = pl.kernel(
        _agg_body,
        out_type=jax.ShapeDtypeStruct((NP, H), jnp.float32),
        mesh=mesh,
        compiler_params=pltpu.CompilerParams(use_tc_tiling_on_sc=False, needs_layout_passes=False),
        scratch_types=[
            pltpu.VMEM((2, _CHUNK_B), jnp.int32),
            pltpu.VMEM((2, _CHUNK_B), jnp.int32),
            pltpu.VMEM((_CHUNK_B,), jnp.int32),
            pltpu.VMEM((_CHUNK_B,), jnp.int32),
            pltpu.VMEM((_CHUNK_B,), jnp.int32),
            pltpu.VMEM((_CHUNK_B,), jnp.int32),
            pltpu.VMEM((_CHUNK_B,), jnp.float32),
            pltpu.VMEM((_CHUNK_B,), jnp.float32),
            pltpu.VMEM((_CHUNK_B, _HH), jnp.float32),
            pltpu.VMEM((_CHUNK_B, _HH), jnp.float32),
            pltpu.VMEM((_CHUNK_B,), jnp.float32),
            pltpu.VMEM((_CHUNK_B,), jnp.float32),
            pltpu.VMEM((_RCH, _HH), jnp.float32),
            pltpu.VMEM((_RCH,), jnp.float32),
            pltpu.VMEM((NT * 16,), jnp.float32),
            pltpu.VMEM_SHARED((NP, _HH), jnp.float32),
            pltpu.VMEM_SHARED((NP,), jnp.float32),
            pltpu.SemaphoreType.DMA,
            pltpu.SemaphoreType.DMA,
            pltpu.SemaphoreType.DMA,
            pltpu.SemaphoreType.DMA,
            pltpu.SemaphoreType.DMA,
        ],
    )
    return kern(scores, pmax, edge_index, vflat)


def kernel(x, edge_index, W_emb, b_emb, Wq, bq, Wk, bk, Wv, bv, Ws, bs):
    wcat = jnp.concatenate([Wq[1:], Wk[1:], Wv[1:], Ws[1:]], axis=1)
    ts = np.linspace(0.0, 1.0, N_STEPS).astype(np.float32)
    bcats = [
        jnp.concatenate(
            [bq + float(t) * Wq[0], bk + float(t) * Wk[0],
             bv + float(t) * Wv[0], bs + float(t) * Ws[0]]
        )
        for t in ts[:N_STEPS - 1]
    ]

    h, q, k, v0, v1, s = _embed_qkvs(x, W_emb, b_emb, wcat, bcats[0])
    ys = [h]
    y = h
    for i in range(N_STEPS - 1):
        dt = float(ts[i + 1] - ts[i])
        vflat = jnp.concatenate([v0, v1], axis=0)
        scores, pmax = _sc_scores(q, k, edge_index)
        aggf = _sc_aggregate(scores, pmax, edge_index, vflat)
        if i < N_STEPS - 2:
            y, q, k, v0, v1, s = _update_qkvs(y, aggf, s, dt, wcat, bcats[i + 1])
        else:
            y = _euler_update(y, aggf, s, dt)
        ys.append(y)
    return jnp.stack(ys, axis=0)


# speedup vs baseline: 12.0265x; 1.0171x over previous
import jax
import jax.numpy as jnp
import numpy as np
from jax import lax
from jax.experimental import pallas as pl
from jax.experimental.pallas import tpu as pltpu
from jax.experimental.pallas import tpu_sc as plsc

N = 10000
E = 320000
D_IN = 128
H = 256
N_STEPS = 4
NP = 10240
NC, NS = 2, 16
NT = NC * NS
SCALE = 0.0625


def _matmul_bias(xa, w, b, bm=2000):
    n, kd = xa.shape
    m = w.shape[1]

    def body(x_ref, w_ref, b_ref, o_ref):
        o_ref[...] = (
            jnp.dot(x_ref[...], w_ref[...], preferred_element_type=jnp.float32)
            + b_ref[...]
        )

    return pl.pallas_call(
        body,
        grid=(n // bm,),
        in_specs=[
            pl.BlockSpec((bm, kd), lambda i: (i, 0)),
            pl.BlockSpec((kd, m), lambda i: (0, 0)),
            pl.BlockSpec((1, m), lambda i: (0, 0)),
        ],
        out_specs=pl.BlockSpec((bm, m), lambda i: (i, 0)),
        out_shape=jax.ShapeDtypeStruct((n, m), jnp.float32),
    )(xa, w, b.reshape(1, m))


_QKVS_OUT_SPECS = [
    pl.BlockSpec((2000, H), lambda i: (i, 0)),
    pl.BlockSpec((2000, H), lambda i: (i, 0)),
    pl.BlockSpec((2000, H // 2), lambda i: (i, 0)),
    pl.BlockSpec((2000, H // 2), lambda i: (i, 0)),
    pl.BlockSpec((2000, H), lambda i: (i, 0)),
]
_QKVS_OUT_SHAPES = [
    jax.ShapeDtypeStruct((N, H), jnp.float32),
    jax.ShapeDtypeStruct((N, H), jnp.float32),
    jax.ShapeDtypeStruct((N, H // 2), jnp.float32),
    jax.ShapeDtypeStruct((N, H // 2), jnp.float32),
    jax.ShapeDtypeStruct((N, H), jnp.float32),
]


def _write_qkvs(acc, q_ref, k_ref, v0_ref, v1_ref, s_ref):
    q_ref[...] = acc[:, 0:256]
    k_ref[...] = acc[:, 256:512]
    v0_ref[...] = acc[:, 512:640]
    v1_ref[...] = acc[:, 640:768]
    s_ref[...] = acc[:, 768:1024]


def _embed_qkvs(x, W_emb, b_emb, wcat, bcat, bm=2000):
    def body(x_ref, we_ref, be_ref, w_ref, b_ref, h_ref,
             q_ref, k_ref, v0_ref, v1_ref, s_ref):
        hblk = (
            jnp.dot(x_ref[...], we_ref[...], preferred_element_type=jnp.float32)
            + be_ref[...]
        )
        h_ref[...] = hblk
        acc = (
            jnp.dot(hblk, w_ref[...], preferred_element_type=jnp.float32)
            + b_ref[...]
        )
        _write_qkvs(acc, q_ref, k_ref, v0_ref, v1_ref, s_ref)

    return pl.pallas_call(
        body,
        grid=(N // bm,),
        in_specs=[
            pl.BlockSpec((bm, D_IN), lambda i: (i, 0)),
            pl.BlockSpec((D_IN, H), lambda i: (0, 0)),
            pl.BlockSpec((1, H), lambda i: (0, 0)),
            pl.BlockSpec((H, 4 * H), lambda i: (0, 0)),
            pl.BlockSpec((1, 4 * H), lambda i: (0, 0)),
        ],
        out_specs=[pl.BlockSpec((bm, H), lambda i: (i, 0))] + _QKVS_OUT_SPECS,
        out_shape=[jax.ShapeDtypeStruct((N, H), jnp.float32)] + _QKVS_OUT_SHAPES,
    )(x, W_emb, b_emb.reshape(1, H), wcat, bcat.reshape(1, 4 * H))


def _update_qkvs(y, aggf, s, dt, wcat, bcat, bm=2000):
    def body(y_ref, a_ref, s_ref, w_ref, b_ref, yn_ref,
             q_ref, k_ref, v0_ref, v1_ref, sn_ref):
        ynew = y_ref[...] + dt * (a_ref[...] + s_ref[...])
        yn_ref[...] = ynew
        acc = (
            jnp.dot(ynew, w_ref[...], preferred_element_type=jnp.float32)
            + b_ref[...]
        )
        _write_qkvs(acc, q_ref, k_ref, v0_ref, v1_ref, sn_ref)

    return pl.pallas_call(
        body,
        grid=(N // bm,),
        in_specs=[
            pl.BlockSpec((bm, H), lambda i: (i, 0)),
            pl.BlockSpec((bm, H), lambda i: (i, 0)),
            pl.BlockSpec((bm, H), lambda i: (i, 0)),
            pl.BlockSpec((H, 4 * H), lambda i: (0, 0)),
            pl.BlockSpec((1, 4 * H), lambda i: (0, 0)),
        ],
        out_specs=[pl.BlockSpec((bm, H), lambda i: (i, 0))] + _QKVS_OUT_SPECS,
        out_shape=[jax.ShapeDtypeStruct((N, H), jnp.float32)] + _QKVS_OUT_SHAPES,
    )(y, aggf, s, wcat, bcat.reshape(1, 4 * H))


def _euler_update(y, aggf, s, dt, bm=2000):
    def body(y_ref, a_ref, s_ref, o_ref):
        o_ref[...] = y_ref[...] + dt * (a_ref[...] + s_ref[...])

    return pl.pallas_call(
        body,
        grid=(N // bm,),
        in_specs=[pl.BlockSpec((bm, H), lambda i: (i, 0))] * 3,
        out_specs=pl.BlockSpec((bm, H), lambda i: (i, 0)),
        out_shape=jax.ShapeDtypeStruct((N, H), jnp.float32),
    )(y, aggf, s)


_CHUNK_A = 80
_EPT_A = E // NT
_CHUNK_B = 80
_EPT_B = E // NS
_RPT = NP // NS
_RCH = 64
_HH = H // 2


def _score_body(q_hbm, k_hbm, ei_hbm, scores_hbm, pmax_hbm,
                ei0, ei1, qr0, qr1, kr0, kr1, sv0, sv1, mx_v,
                sem0, sem1, semi0, semi1, semw):
    wid = lax.axis_index("s") * NC + lax.axis_index("c")
    base = wid * _EPT_A
    lanes = lax.iota(jnp.int32, 16)
    ones = jnp.full((16,), 1, jnp.int32)
    eib = (ei0, ei1)
    qrb = (qr0, qr1)
    krb = (kr0, kr1)
    svb = (sv0, sv1)
    semb = (sem0, sem1)
    semi = (semi0, semi1)
    nch = _EPT_A // _CHUNK_A

    def fire_idx(cur, b):
        off = base + cur * _CHUNK_A
        pltpu.async_copy(ei_hbm.at[:, pl.ds(off, _CHUNK_A)], eib[b], semi[b])

    def wait_idx(b):
        pltpu.make_async_copy(ei_hbm.at[:, pl.ds(0, _CHUNK_A)], eib[b],
                              semi[b]).wait()

    def fire_gathers(b):
        pltpu.async_copy(q_hbm.at[eib[b].at[1]], qrb[b], semb[b])
        pltpu.async_copy(k_hbm.at[eib[b].at[0]], krb[b], semb[b])

    def wait_gathers(b):
        pltpu.make_async_copy(q_hbm.at[pl.ds(0, _CHUNK_A)], qrb[b], semb[b]).wait()
        pltpu.make_async_copy(k_hbm.at[pl.ds(0, _CHUNK_A)], krb[b], semb[b]).wait()

    def compute(cur, b, mv):
        off = base + cur * _CHUNK_A

        @pl.when(cur >= 2)
        def _():
            pltpu.make_async_copy(svb[b], scores_hbm.at[pl.ds(0, _CHUNK_A)],
                                  semw).wait()

        def grp(g, mcur):
            eidx = lanes + g * 16
            zf = jnp.zeros((16,), jnp.float32)

            def jblk(jb, carry):
                cidx = carry[0]
                accs = list(carry[1:])
                for u in range(16):
                    qv = plsc.load_gather(qrb[b], [eidx, cidx])
                    kv = plsc.load_gather(krb[b], [eidx, cidx])
                    accs[u % 8] = accs[u % 8] + qv * kv
                    cidx = cidx + ones
                return (cidx, *accs)

            carry = lax.fori_loop(
                0, 15, jblk, (lanes,) + (zf,) * 8)
            cidx = carry[0]
            accs = list(carry[1:])
            cmod = jnp.full((16,), H, jnp.int32)
            for u in range(16):
                cw = jnp.where(cidx >= cmod, cidx - cmod, cidx)
                qv = plsc.load_gather(qrb[b], [eidx, cw])
                kv = plsc.load_gather(krb[b], [eidx, cw])
                accs[u % 8] = accs[u % 8] + qv * kv
                cidx = cidx + ones
            sc = (((accs[0] + accs[1]) + (accs[2] + accs[3]))
                  + ((accs[4] + accs[5]) + (accs[6] + accs[7]))) * SCALE
            svb[b][pl.ds(g * 16, 16)] = sc
            return jnp.maximum(mcur, sc)

        mv = lax.fori_loop(0, _CHUNK_A // 16, grp, mv)
        pltpu.async_copy(svb[b], scores_hbm.at[pl.ds(off, _CHUNK_A)], semw)
        return mv

    fire_idx(0, 0)
    fire_idx(1, 1)
    wait_idx(0)
    fire_gathers(0)

    def body(k2, mv):
        for b in (0, 1):
            cur = 2 * k2 + b
            wait_gathers(b)

            @pl.when(cur + 2 < nch)
            def _():
                fire_idx(cur + 2, b)

            wait_idx(1 - b)
            fire_gathers(1 - b)
            mv = compute(cur, b, mv)
        return mv

    mv = lax.fori_loop(0, (nch - 1) // 2, body,
                       jnp.full((16,), -3.0e38, jnp.float32))
    wait_gathers(0)
    mv = compute(nch - 1, 0, mv)
    pltpu.make_async_copy(sv0, scores_hbm.at[pl.ds(0, _CHUNK_A)], semw).wait()
    pltpu.make_async_copy(sv1, scores_hbm.at[pl.ds(0, _CHUNK_A)], semw).wait()
    mx_v[...] = mv
    pltpu.sync_copy(mx_v, pmax_hbm.at[pl.ds(wid * 16, 16)])


def _sc_scores(q, k, edge_index):
    mesh = plsc.VectorSubcoreMesh(core_axis_name="c", subcore_axis_name="s",
                                  num_cores=NC, num_subcores=NS)
    kern = pl.kernel(
        _score_body,
        out_type=[
            jax.ShapeDtypeStruct((E,), jnp.float32),
            jax.ShapeDtypeStruct((NT * 16,), jnp.float32),
        ],
        mesh=mesh,
        compiler_params=pltpu.CompilerParams(use_tc_tiling_on_sc=False, needs_layout_passes=False),
        scratch_types=[
            pltpu.VMEM((2, _CHUNK_A), jnp.int32),
            pltpu.VMEM((2, _CHUNK_A), jnp.int32),
            pltpu.VMEM((_CHUNK_A, H), jnp.float32),
            pltpu.VMEM((_CHUNK_A, H), jnp.float32),
            pltpu.VMEM((_CHUNK_A, H), jnp.float32),
            pltpu.VMEM((_CHUNK_A, H), jnp.float32),
            pltpu.VMEM((_CHUNK_A,), jnp.float32),
            pltpu.VMEM((_CHUNK_A,), jnp.float32),
            pltpu.VMEM((16,), jnp.float32),
            pltpu.SemaphoreType.DMA,
            pltpu.SemaphoreType.DMA,
            pltpu.SemaphoreType.DMA,
            pltpu.SemaphoreType.DMA,
            pltpu.SemaphoreType.DMA,
        ],
    )
    return kern(q, k, edge_index)


def _agg_body(scores_hbm, pmax_hbm, ei_hbm, vflat_hbm, out_hbm,
              ei0, ei1, dst0, dst1, ix0, ix1, ev0, ev1, vr0, vr1,
              sc0, sc1, arows, den_v, pm_v, agg_sh, den_sh,
              semg0, semg1, semi0, semi1, sems):
    c = lax.axis_index("c")
    t = lax.axis_index("s")
    eib = (ei0, ei1)
    dstb = (dst0, dst1)
    ixb = (ix0, ix1)
    evb = (ev0, ev1)
    vrb = (vr0, vr1)
    scb = (sc0, sc1)
    semg = (semg0, semg1)
    semi = (semi0, semi1)

    def zrow(i, _):
        for j in range(_HH // 16):
            arows[i, pl.ds(j * 16, 16)] = jnp.zeros((16,), jnp.float32)
        return 0

    lax.fori_loop(0, _RCH, zrow, 0)

    def zev(g, _):
        ev0[pl.ds(g * 16, 16)] = jnp.zeros((16,), jnp.float32)
        return 0

    lax.fori_loop(0, _CHUNK_B // 16, zev, 0)

    def zagg(rc, _):
        pltpu.sync_copy(arows, agg_sh.at[pl.ds(t * _RPT + rc * _RCH, _RCH)])
        return 0

    lax.fori_loop(0, _RPT // _RCH, zagg, 0)

    def zden(zi, _):
        pltpu.sync_copy(ev0, den_sh.at[pl.ds(t * _RPT + zi * _CHUNK_B, _CHUNK_B)])
        return 0

    lax.fori_loop(0, _RPT // _CHUNK_B, zden, 0)
    plsc.subcore_barrier()

    pltpu.sync_copy(pmax_hbm, pm_v)
    mv = pm_v[pl.ds(0, 16)]
    for g in range(1, NT):
        mv = jnp.maximum(mv, pm_v[pl.ds(g * 16, 16)])
    gmax = jnp.max(mv)

    ebase = t * _EPT_B
    coff = jnp.full((16,), c * N, jnp.int32)
    nch = _EPT_B // _CHUNK_B

    def fire_idx(cur, b):
        off = ebase + cur * _CHUNK_B
        pltpu.async_copy(ei_hbm.at[:, pl.ds(off, _CHUNK_B)], eib[b], semi[b])
        pltpu.async_copy(scores_hbm.at[pl.ds(off, _CHUNK_B)], scb[b], semi[b])

    def wait_idx(b):
        pltpu.make_async_copy(ei_hbm.at[:, pl.ds(0, _CHUNK_B)], eib[b],
                              semi[b]).wait()
        pltpu.make_async_copy(scores_hbm.at[pl.ds(0, _CHUNK_B)], scb[b],
                              semi[b]).wait()

    def vec_compute(b):
        def vec(g, _):
            sl = pl.ds(g * 16, 16)
            evb[b][sl] = jnp.exp(scb[b][sl] - gmax)
            ixb[b][sl] = eib[b][0, sl] + coff
            dstb[b][sl] = eib[b][1, sl]
            return 0

        lax.fori_loop(0, _CHUNK_B // 16, vec, 0)

    def fire_gather(b):
        pltpu.async_copy(vflat_hbm.at[ixb[b]], vrb[b], semg[b])

    def wait_gather(b):
        pltpu.make_async_copy(vflat_hbm.at[pl.ds(0, _CHUNK_B)], vrb[b],
                              semg[b]).wait()

    def wait_scatter_pair():
        pltpu.make_async_copy(vr0, agg_sh.at[pl.ds(0, _CHUNK_B)], sems).wait()
        pltpu.make_async_copy(ev0, den_sh.at[pl.ds(0, _CHUNK_B)], sems).wait()

    def process(b):
        wait_gather(b)

        def edge(e4, _):
            e = e4 * 4
            ebs = [plsc.load_gather(evb[b], [jnp.full((16,), e + u, jnp.int32)])
                   for u in range(4)]
            for j in range(_HH // 16):
                sl = pl.ds(j * 16, 16)
                for u in range(4):
                    vrb[b][e + u, sl] = vrb[b][e + u, sl] * ebs[u]
            return 0

        lax.fori_loop(0, _CHUNK_B // 4, edge, 0)
        pltpu.async_copy(vrb[b], agg_sh.at[dstb[b]], sems, add=True)
        pltpu.async_copy(evb[b], den_sh.at[dstb[b]], sems, add=True)

    fire_idx(0, 0)
    fire_idx(1, 1)
    wait_idx(0)
    vec_compute(0)
    fire_idx(2, 0)
    fire_gather(0)

    def body(k2, _):
        for b2 in (0, 1):
            ci = 2 * k2 + b2

            @pl.when(ci >= 1)
            def _():
                wait_scatter_pair()

            wait_idx(1 - b2)
            vec_compute(1 - b2)

            @pl.when(ci + 3 < nch)
            def _():
                fire_idx(ci + 3, 1 - b2)

            fire_gather(1 - b2)
            process(b2)
        return 0

    lax.fori_loop(0, (nch - 2) // 2, body, 0)
    wait_scatter_pair()
    wait_idx(1)
    vec_compute(1)
    fire_gather(1)
    process(0)
    wait_scatter_pair()
    process(1)
    wait_scatter_pair()
    plsc.subcore_barrier()

    rbase = t * _RPT

    def nchunk(rc, _):
        r0 = rbase + rc * _RCH
        pltpu.sync_copy(agg_sh.at[pl.ds(r0, _RCH)], arows)
        pltpu.sync_copy(den_sh.at[pl.ds(r0, _RCH)], den_v)

        def row(r, _):
            db = plsc.load_gather(den_v, [jnp.full((16,), r, jnp.int32)]) + 1e-16
            for j in range(_HH // 16):
                sl = pl.ds(j * 16, 16)
                arows[r, sl] = arows[r, sl] / db
            return 0

        lax.fori_loop(0, _RCH, row, 0)
        pltpu.sync_copy(arows, out_hbm.at[pl.ds(r0, _RCH), pl.ds(c * _HH, _HH)])
        return 0

    lax.fori_loop(0, _RPT // _RCH, nchunk, 0)


def _sc_aggregate(scores, pmax, edge_index, vflat):
    mesh = plsc.VectorSubcoreMesh(core_axis_name="c", subcore_axis_name="s",
                                  num_cores=NC, num_subcores=NS)
    kern = pl.kernel(
        _agg_body,
        out_type=jax.ShapeDtypeStruct((NP, H), jnp.float32),
        mesh=mesh,
        compiler_params=pltpu.CompilerParams(use_tc_tiling_on_sc=False, needs_layout_passes=False),
        scratch_types=[
            pltpu.VMEM((2, _CHUNK_B), jnp.int32),
            pltpu.VMEM((2, _CHUNK_B), jnp.int32),
            pltpu.VMEM((_CHUNK_B,), jnp.int32),
            pltpu.VMEM((_CHUNK_B,), jnp.int32),
            pltpu.VMEM((_CHUNK_B,), jnp.int32),
            pltpu.VMEM((_CHUNK_B,), jnp.int32),
            pltpu.VMEM((_CHUNK_B,), jnp.float32),
            pltpu.VMEM((_CHUNK_B,), jnp.float32),
            pltpu.VMEM((_CHUNK_B, _HH), jnp.float32),
            pltpu.VMEM((_CHUNK_B, _HH), jnp.float32),
            pltpu.VMEM((_CHUNK_B,), jnp.float32),
            pltpu.VMEM((_CHUNK_B,), jnp.float32),
            pltpu.VMEM((_RCH, _HH), jnp.float32),
            pltpu.VMEM((_RCH,), jnp.float32),
            pltpu.VMEM((NT * 16,), jnp.float32),
            pltpu.VMEM_SHARED((NP, _HH), jnp.float32),
            pltpu.VMEM_SHARED((NP,), jnp.float32),
            pltpu.SemaphoreType.DMA,
            pltpu.SemaphoreType.DMA,
            pltpu.SemaphoreType.DMA,
            pltpu.SemaphoreType.DMA,
            pltpu.SemaphoreType.DMA,
        ],
    )
    return kern(scores, pmax, edge_index, vflat)


def kernel(x, edge_index, W_emb, b_emb, Wq, bq, Wk, bk, Wv, bv, Ws, bs):
    wcat = jnp.concatenate([Wq[1:], Wk[1:], Wv[1:], Ws[1:]], axis=1)
    ts = np.linspace(0.0, 1.0, N_STEPS).astype(np.float32)
    bcats = [
        jnp.concatenate(
            [bq + float(t) * Wq[0], bk + float(t) * Wk[0],
             bv + float(t) * Wv[0], bs + float(t) * Ws[0]]
        )
        for t in ts[:N_STEPS - 1]
    ]

    h, q, k, v0, v1, s = _embed_qkvs(x, W_emb, b_emb, wcat, bcats[0])
    ys = [h]
    y = h
    for i in range(N_STEPS - 1):
        dt = float(ts[i + 1] - ts[i])
        vflat = jnp.concatenate([v0, v1], axis=0)
        scores, pmax = _sc_scores(q, k, edge_index)
        aggf = _sc_aggregate(scores, pmax, edge_index, vflat)
        if i < N_STEPS - 2:
            y, q, k, v0, v1, s = _update_qkvs(y, aggf, s, dt, wcat, bcats[i + 1])
        else:
            y = _euler_update(y, aggf, s, dt)
        ys.append(y)
    return jnp.stack(ys, axis=0)


# speedup vs baseline: 12.0278x; 1.0001x over previous
import jax
import jax.numpy as jnp
import numpy as np
from jax import lax
from jax.experimental import pallas as pl
from jax.experimental.pallas import tpu as pltpu
from jax.experimental.pallas import tpu_sc as plsc

N = 10000
E = 320000
D_IN = 128
H = 256
N_STEPS = 4
NP = 10240
NC, NS = 2, 16
NT = NC * NS
SCALE = 0.0625


_QKVS_OUT_SPECS = [
    pl.BlockSpec((2000, H), lambda i: (i, 0)),
    pl.BlockSpec((2000, H), lambda i: (i, 0)),
    pl.BlockSpec((2000, H // 2), lambda i: (i, 0)),
    pl.BlockSpec((2000, H // 2), lambda i: (i, 0)),
    pl.BlockSpec((2000, H), lambda i: (i, 0)),
]
_QKVS_OUT_SHAPES = [
    jax.ShapeDtypeStruct((N, H), jnp.float32),
    jax.ShapeDtypeStruct((N, H), jnp.float32),
    jax.ShapeDtypeStruct((N, H // 2), jnp.float32),
    jax.ShapeDtypeStruct((N, H // 2), jnp.float32),
    jax.ShapeDtypeStruct((N, H), jnp.float32),
]


def _write_qkvs(acc, q_ref, k_ref, v0_ref, v1_ref, s_ref):
    q_ref[...] = acc[:, 0:256]
    k_ref[...] = acc[:, 256:512]
    v0_ref[...] = acc[:, 512:640]
    v1_ref[...] = acc[:, 640:768]
    s_ref[...] = acc[:, 768:1024]


def _embed_qkvs(x, W_emb, b_emb, wcat, bcat, bm=2000):
    def body(x_ref, we_ref, be_ref, w_ref, b_ref, h_ref,
             q_ref, k_ref, v0_ref, v1_ref, s_ref):
        hblk = (
            jnp.dot(x_ref[...], we_ref[...], preferred_element_type=jnp.float32)
            + be_ref[...]
        )
        h_ref[...] = hblk
        acc = (
            jnp.dot(hblk, w_ref[...], preferred_element_type=jnp.float32)
            + b_ref[...]
        )
        _write_qkvs(acc, q_ref, k_ref, v0_ref, v1_ref, s_ref)

    return pl.pallas_call(
        body,
        grid=(N // bm,),
        in_specs=[
            pl.BlockSpec((bm, D_IN), lambda i: (i, 0)),
            pl.BlockSpec((D_IN, H), lambda i: (0, 0)),
            pl.BlockSpec((1, H), lambda i: (0, 0)),
            pl.BlockSpec((H, 4 * H), lambda i: (0, 0)),
            pl.BlockSpec((1, 4 * H), lambda i: (0, 0)),
        ],
        out_specs=[pl.BlockSpec((bm, H), lambda i: (i, 0))] + _QKVS_OUT_SPECS,
        out_shape=[jax.ShapeDtypeStruct((N, H), jnp.float32)] + _QKVS_OUT_SHAPES,
    )(x, W_emb, b_emb.reshape(1, H), wcat, bcat.reshape(1, 4 * H))


def _update_qkvs(y, aggf, s, dt, wcat, bcat, bm=2000):
    def body(y_ref, a_ref, s_ref, w_ref, b_ref, yn_ref,
             q_ref, k_ref, v0_ref, v1_ref, sn_ref):
        ynew = y_ref[...] + dt * (a_ref[...] + s_ref[...])
        yn_ref[...] = ynew
        acc = (
            jnp.dot(ynew, w_ref[...], preferred_element_type=jnp.float32)
            + b_ref[...]
        )
        _write_qkvs(acc, q_ref, k_ref, v0_ref, v1_ref, sn_ref)

    return pl.pallas_call(
        body,
        grid=(N // bm,),
        in_specs=[
            pl.BlockSpec((bm, H), lambda i: (i, 0)),
            pl.BlockSpec((bm, H), lambda i: (i, 0)),
            pl.BlockSpec((bm, H), lambda i: (i, 0)),
            pl.BlockSpec((H, 4 * H), lambda i: (0, 0)),
            pl.BlockSpec((1, 4 * H), lambda i: (0, 0)),
        ],
        out_specs=[pl.BlockSpec((bm, H), lambda i: (i, 0))] + _QKVS_OUT_SPECS,
        out_shape=[jax.ShapeDtypeStruct((N, H), jnp.float32)] + _QKVS_OUT_SHAPES,
    )(y, aggf, s, wcat, bcat.reshape(1, 4 * H))


def _euler_update(y, aggf, s, dt, bm=2000):
    def body(y_ref, a_ref, s_ref, o_ref):
        o_ref[...] = y_ref[...] + dt * (a_ref[...] + s_ref[...])

    return pl.pallas_call(
        body,
        grid=(N // bm,),
        in_specs=[pl.BlockSpec((bm, H), lambda i: (i, 0))] * 3,
        out_specs=pl.BlockSpec((bm, H), lambda i: (i, 0)),
        out_shape=jax.ShapeDtypeStruct((N, H), jnp.float32),
    )(y, aggf, s)


_CHUNK_A = 80
_EPT_A = E // NT
_CHUNK_B = 80
_EPT_B = E // NS
_RPT = NP // NS
_RCH = 64
_HH = H // 2


def _score_body(q_hbm, k_hbm, ei_hbm, scores_hbm, pmax_hbm,
                ei0, ei1, qr0, qr1, kr0, kr1, sv0, sv1, mx_v,
                sem0, sem1, semi0, semi1, semw):
    wid = lax.axis_index("s") * NC + lax.axis_index("c")
    base = wid * _EPT_A
    lanes = lax.iota(jnp.int32, 16)
    ones = jnp.full((16,), 1, jnp.int32)
    eib = (ei0, ei1)
    qrb = (qr0, qr1)
    krb = (kr0, kr1)
    svb = (sv0, sv1)
    semb = (sem0, sem1)
    semi = (semi0, semi1)
    nch = _EPT_A // _CHUNK_A

    def fire_idx(cur, b):
        off = base + cur * _CHUNK_A
        pltpu.async_copy(ei_hbm.at[:, pl.ds(off, _CHUNK_A)], eib[b], semi[b])

    def wait_idx(b):
        pltpu.make_async_copy(ei_hbm.at[:, pl.ds(0, _CHUNK_A)], eib[b],
                              semi[b]).wait()

    def fire_gathers(b):
        pltpu.async_copy(q_hbm.at[eib[b].at[1]], qrb[b], semb[b])
        pltpu.async_copy(k_hbm.at[eib[b].at[0]], krb[b], semb[b])

    def wait_gathers(b):
        pltpu.make_async_copy(q_hbm.at[pl.ds(0, _CHUNK_A)], qrb[b], semb[b]).wait()
        pltpu.make_async_copy(k_hbm.at[pl.ds(0, _CHUNK_A)], krb[b], semb[b]).wait()

    def compute(cur, b, mv):
        off = base + cur * _CHUNK_A

        @pl.when(cur >= 2)
        def _():
            pltpu.make_async_copy(svb[b], scores_hbm.at[pl.ds(0, _CHUNK_A)],
                                  semw).wait()

        def grp(g, mcur):
            eidx = lanes + g * 16
            zf = jnp.zeros((16,), jnp.float32)

            def jblk(jb, carry):
                cidx = carry[0]
                accs = list(carry[1:])
                for u in range(16):
                    qv = plsc.load_gather(qrb[b], [eidx, cidx])
                    kv = plsc.load_gather(krb[b], [eidx, cidx])
                    accs[u % 8] = accs[u % 8] + qv * kv
                    cidx = cidx + ones
                return (cidx, *accs)

            carry = lax.fori_loop(
                0, 15, jblk, (lanes,) + (zf,) * 8)
            cidx = carry[0]
            accs = list(carry[1:])
            cmod = jnp.full((16,), H, jnp.int32)
            for u in range(16):
                cw = jnp.where(cidx >= cmod, cidx - cmod, cidx)
                qv = plsc.load_gather(qrb[b], [eidx, cw])
                kv = plsc.load_gather(krb[b], [eidx, cw])
                accs[u % 8] = accs[u % 8] + qv * kv
                cidx = cidx + ones
            sc = (((accs[0] + accs[1]) + (accs[2] + accs[3]))
                  + ((accs[4] + accs[5]) + (accs[6] + accs[7]))) * SCALE
            svb[b][pl.ds(g * 16, 16)] = sc
            return jnp.maximum(mcur, sc)

        mv = lax.fori_loop(0, _CHUNK_A // 16, grp, mv)
        pltpu.async_copy(svb[b], scores_hbm.at[pl.ds(off, _CHUNK_A)], semw)
        return mv

    fire_idx(0, 0)
    fire_idx(1, 1)
    wait_idx(0)
    fire_gathers(0)

    def body(k2, mv):
        for b in (0, 1):
            cur = 2 * k2 + b
            wait_gathers(b)

            @pl.when(cur + 2 < nch)
            def _():
                fire_idx(cur + 2, b)

            wait_idx(1 - b)
            fire_gathers(1 - b)
            mv = compute(cur, b, mv)
        return mv

    mv = lax.fori_loop(0, (nch - 1) // 2, body,
                       jnp.full((16,), -3.0e38, jnp.float32))
    wait_gathers(0)
    mv = compute(nch - 1, 0, mv)
    pltpu.make_async_copy(sv0, scores_hbm.at[pl.ds(0, _CHUNK_A)], semw).wait()
    pltpu.make_async_copy(sv1, scores_hbm.at[pl.ds(0, _CHUNK_A)], semw).wait()
    mx_v[...] = mv
    pltpu.sync_copy(mx_v, pmax_hbm.at[pl.ds(wid * 16, 16)])


def _sc_scores(q, k, edge_index):
    mesh = plsc.VectorSubcoreMesh(core_axis_name="c", subcore_axis_name="s",
                                  num_cores=NC, num_subcores=NS)
    kern = pl.kernel(
        _score_body,
        out_type=[
            jax.ShapeDtypeStruct((E,), jnp.float32),
            jax.ShapeDtypeStruct((NT * 16,), jnp.float32),
        ],
        mesh=mesh,
        compiler_params=pltpu.CompilerParams(use_tc_tiling_on_sc=False, needs_layout_passes=False),
        scratch_types=[
            pltpu.VMEM((2, _CHUNK_A), jnp.int32),
            pltpu.VMEM((2, _CHUNK_A), jnp.int32),
            pltpu.VMEM((_CHUNK_A, H), jnp.float32),
            pltpu.VMEM((_CHUNK_A, H), jnp.float32),
            pltpu.VMEM((_CHUNK_A, H), jnp.float32),
            pltpu.VMEM((_CHUNK_A, H), jnp.float32),
            pltpu.VMEM((_CHUNK_A,), jnp.float32),
            pltpu.VMEM((_CHUNK_A,), jnp.float32),
            pltpu.VMEM((16,), jnp.float32),
            pltpu.SemaphoreType.DMA,
            pltpu.SemaphoreType.DMA,
            pltpu.SemaphoreType.DMA,
            pltpu.SemaphoreType.DMA,
            pltpu.SemaphoreType.DMA,
        ],
    )
    return kern(q, k, edge_index)


def _agg_body(scores_hbm, pmax_hbm, ei_hbm, vflat_hbm, out_hbm,
              ei0, ei1, dst0, dst1, ix0, ix1, ev0, ev1, vr0, vr1,
              sc0, sc1, arows, den_v, pm_v, agg_sh, den_sh,
              semg0, semg1, semi0, semi1, sems):
    c = lax.axis_index("c")
    t = lax.axis_index("s")
    eib = (ei0, ei1)
    dstb = (dst0, dst1)
    ixb = (ix0, ix1)
    evb = (ev0, ev1)
    vrb = (vr0, vr1)
    scb = (sc0, sc1)
    semg = (semg0, semg1)
    semi = (semi0, semi1)

    def zrow(i, _):
        for j in range(_HH // 16):
            arows[i, pl.ds(j * 16, 16)] = jnp.zeros((16,), jnp.float32)
        return 0

    lax.fori_loop(0, _RCH, zrow, 0)

    def zev(g, _):
        ev0[pl.ds(g * 16, 16)] = jnp.zeros((16,), jnp.float32)
        return 0

    lax.fori_loop(0, _CHUNK_B // 16, zev, 0)

    def zagg(rc, _):
        pltpu.sync_copy(arows, agg_sh.at[pl.ds(t * _RPT + rc * _RCH, _RCH)])
        return 0

    lax.fori_loop(0, _RPT // _RCH, zagg, 0)

    def zden(zi, _):
        pltpu.sync_copy(ev0, den_sh.at[pl.ds(t * _RPT + zi * _CHUNK_B, _CHUNK_B)])
        return 0

    lax.fori_loop(0, _RPT // _CHUNK_B, zden, 0)
    plsc.subcore_barrier()

    pltpu.sync_copy(pmax_hbm, pm_v)
    mv = pm_v[pl.ds(0, 16)]
    for g in range(1, NT):
        mv = jnp.maximum(mv, pm_v[pl.ds(g * 16, 16)])
    gmax = jnp.max(mv)

    ebase = t * _EPT_B
    coff = jnp.full((16,), c * N, jnp.int32)
    nch = _EPT_B // _CHUNK_B

    def fire_idx(cur, b):
        off = ebase + cur * _CHUNK_B
        pltpu.async_copy(ei_hbm.at[:, pl.ds(off, _CHUNK_B)], eib[b], semi[b])
        pltpu.async_copy(scores_hbm.at[pl.ds(off, _CHUNK_B)], scb[b], semi[b])

    def wait_idx(b):
        pltpu.make_async_copy(ei_hbm.at[:, pl.ds(0, _CHUNK_B)], eib[b],
                              semi[b]).wait()
        pltpu.make_async_copy(scores_hbm.at[pl.ds(0, _CHUNK_B)], scb[b],
                              semi[b]).wait()

    def vec_compute(b):
        def vec(g, _):
            sl = pl.ds(g * 16, 16)
            evb[b][sl] = jnp.exp(scb[b][sl] - gmax)
            ixb[b][sl] = eib[b][0, sl] + coff
            dstb[b][sl] = eib[b][1, sl]
            return 0

        lax.fori_loop(0, _CHUNK_B // 16, vec, 0)

    def fire_gather(b):
        pltpu.async_copy(vflat_hbm.at[ixb[b]], vrb[b], semg[b])

    def wait_gather(b):
        pltpu.make_async_copy(vflat_hbm.at[pl.ds(0, _CHUNK_B)], vrb[b],
                              semg[b]).wait()

    def wait_scatter_pair():
        pltpu.make_async_copy(vr0, agg_sh.at[pl.ds(0, _CHUNK_B)], sems).wait()
        pltpu.make_async_copy(ev0, den_sh.at[pl.ds(0, _CHUNK_B)], sems).wait()

    def process(b):
        wait_gather(b)

        def edge(e4, _):
            e = e4 * 4
            ebs = [plsc.load_gather(evb[b], [jnp.full((16,), e + u, jnp.int32)])
                   for u in range(4)]
            for j in range(_HH // 16):
                sl = pl.ds(j * 16, 16)
                for u in range(4):
                    vrb[b][e + u, sl] = vrb[b][e + u, sl] * ebs[u]
            return 0

        lax.fori_loop(0, _CHUNK_B // 4, edge, 0)
        pltpu.async_copy(vrb[b], agg_sh.at[dstb[b]], sems, add=True)
        pltpu.async_copy(evb[b], den_sh.at[dstb[b]], sems, add=True)

    fire_idx(0, 0)
    fire_idx(1, 1)
    wait_idx(0)
    vec_compute(0)
    fire_idx(2, 0)
    fire_gather(0)

    def body(k2, _):
        for b2 in (0, 1):
            ci = 2 * k2 + b2

            @pl.when(ci >= 1)
            def _():
                wait_scatter_pair()

            wait_idx(1 - b2)
            vec_compute(1 - b2)

            @pl.when(ci + 3 < nch)
            def _():
                fire_idx(ci + 3, 1 - b2)

            fire_gather(1 - b2)
            process(b2)
        return 0

    lax.fori_loop(0, (nch - 2) // 2, body, 0)
    wait_scatter_pair()
    wait_idx(1)
    vec_compute(1)
    fire_gather(1)
    process(0)
    wait_scatter_pair()
    process(1)
    wait_scatter_pair()
    plsc.subcore_barrier()

    rbase = t * _RPT

    def nchunk(rc, _):
        r0 = rbase + rc * _RCH
        pltpu.sync_copy(agg_sh.at[pl.ds(r0, _RCH)], arows)
        pltpu.sync_copy(den_sh.at[pl.ds(r0, _RCH)], den_v)

        def row(r, _):
            db = plsc.load_gather(den_v, [jnp.full((16,), r, jnp.int32)]) + 1e-16
            for j in range(_HH // 16):
                sl = pl.ds(j * 16, 16)
                arows[r, sl] = arows[r, sl] / db
            return 0

        lax.fori_loop(0, _RCH, row, 0)
        pltpu.sync_copy(arows, out_hbm.at[pl.ds(r0, _RCH), pl.ds(c * _HH, _HH)])
        return 0

    lax.fori_loop(0, _RPT // _RCH, nchunk, 0)


def _sc_aggregate(scores, pmax, edge_index, vflat):
    mesh = plsc.VectorSubcoreMesh(core_axis_name="c", subcore_axis_name="s",
                                  num_cores=NC, num_subcores=NS)
    kern = pl.kernel(
        _agg_body,
        out_type=jax.ShapeDtypeStruct((NP, H), jnp.float32),
        mesh=mesh,
        compiler_params=pltpu.CompilerParams(use_tc_tiling_on_sc=False, needs_layout_passes=False),
        scratch_types=[
            pltpu.VMEM((2, _CHUNK_B), jnp.int32),
            pltpu.VMEM((2, _CHUNK_B), jnp.int32),
            pltpu.VMEM((_CHUNK_B,), jnp.int32),
            pltpu.VMEM((_CHUNK_B,), jnp.int32),
            pltpu.VMEM((_CHUNK_B,), jnp.int32),
            pltpu.VMEM((_CHUNK_B,), jnp.int32),
            pltpu.VMEM((_CHUNK_B,), jnp.float32),
            pltpu.VMEM((_CHUNK_B,), jnp.float32),
            pltpu.VMEM((_CHUNK_B, _HH), jnp.float32),
            pltpu.VMEM((_CHUNK_B, _HH), jnp.float32),
            pltpu.VMEM((_CHUNK_B,), jnp.float32),
            pltpu.VMEM((_CHUNK_B,), jnp.float32),
            pltpu.VMEM((_RCH, _HH), jnp.float32),
            pltpu.VMEM((_RCH,), jnp.float32),
            pltpu.VMEM((NT * 16,), jnp.float32),
            pltpu.VMEM_SHARED((NP, _HH), jnp.float32),
            pltpu.VMEM_SHARED((NP,), jnp.float32),
            pltpu.SemaphoreType.DMA,
            pltpu.SemaphoreType.DMA,
            pltpu.SemaphoreType.DMA,
            pltpu.SemaphoreType.DMA,
            pltpu.SemaphoreType.DMA,
        ],
    )
    return kern(scores, pmax, edge_index, vflat)


def kernel(x, edge_index, W_emb, b_emb, Wq, bq, Wk, bk, Wv, bv, Ws, bs):
    wcat = jnp.concatenate([Wq[1:], Wk[1:], Wv[1:], Ws[1:]], axis=1)
    ts = np.linspace(0.0, 1.0, N_STEPS).astype(np.float32)
    bcats = [
        jnp.concatenate(
            [bq + float(t) * Wq[0], bk + float(t) * Wk[0],
             bv + float(t) * Wv[0], bs + float(t) * Ws[0]]
        )
        for t in ts[:N_STEPS - 1]
    ]

    h, q, k, v0, v1, s = _embed_qkvs(x, W_emb, b_emb, wcat, bcats[0])
    ys = [h]
    y = h
    for i in range(N_STEPS - 1):
        dt = float(ts[i + 1] - ts[i])
        vflat = jnp.concatenate([v0, v1], axis=0)
        scores, pmax = _sc_scores(q, k, edge_index)
        aggf = _sc_aggregate(scores, pmax, edge_index, vflat)
        if i < N_STEPS - 2:
            y, q, k, v0, v1, s = _update_qkvs(y, aggf, s, dt, wcat, bcats[i + 1])
        else:
            y = _euler_update(y, aggf, s, dt)
        ys.append(y)
    return jnp.stack(ys, axis=0)
